# Initial kernel scaffold; baseline (speedup 1.0000x reference)
import jax
import jax.numpy as jnp
from jax.experimental import pallas as pl


def kernel(z, pos, batch, emb, Wm1, bm1, Wm2, bm2, Wc1, Wc2, bc2, Wl, bl, lin1_W, lin1_b, lin2_W, lin2_b):
    raise NotImplementedError("write your pallas kernel here")


# speedup vs baseline: 17.9972x; 17.9972x over previous
import functools

import jax
import jax.numpy as jnp
from jax import lax
from jax.experimental import pallas as pl
from jax.experimental.pallas import tpu as pltpu
from jax.experimental.pallas import tpu_sc as plsc

N = 8192
G = 1024
H = 128
NF = 128
NG = 50
NGP = 64
OUT = 128
NI = 6
CUTOFF = 10.0
K = 512

NC = 2
NS = 16
NW = NC * NS
GPT = G // NW
CU = 32
CV = 32
RT = 256


def _ssp(x):
    return jax.nn.softplus(x) - jnp.log(2.0)


def _tables_body(wm1_ref, bm1_ref, wm2_ref, bm2_ref, t_ref):
    d = lax.broadcasted_iota(jnp.int32, (K, 1), 0).astype(jnp.float32) \
        * (CUTOFF / (K - 1))
    off = lax.broadcasted_iota(jnp.int32, (1, NGP), 1).astype(jnp.float32) \
        * (CUTOFF / (NG - 1))
    step = CUTOFF / (NG - 1)
    coeff = -0.5 / (step * step)
    ea = jnp.exp(coeff * (d - off) ** 2)
    t = _ssp(jnp.dot(ea, wm1_ref[0], preferred_element_type=jnp.float32)
             + bm1_ref[0])
    t = jnp.dot(t, wm2_ref[0], preferred_element_type=jnp.float32) + bm2_ref[0]
    c = 0.5 * (jnp.cos(d * (jnp.pi / CUTOFF)) + 1.0)
    t_ref[0] = t * c


def _make_tables(wm1p, bm1r, wm2, bm2r):
    return pl.pallas_call(
        _tables_body,
        grid=(NI,),
        in_specs=[
            pl.BlockSpec((1, NGP, NF), lambda i: (i, 0, 0)),
            pl.BlockSpec((1, 1, NF), lambda i: (i, 0, 0)),
            pl.BlockSpec((1, NF, NF), lambda i: (i, 0, 0)),
            pl.BlockSpec((1, 1, NF), lambda i: (i, 0, 0)),
        ],
        out_specs=pl.BlockSpec((1, K, NF), lambda i: (i, 0, 0)),
        out_shape=jax.ShapeDtypeStruct((NI, K, NF), jnp.float32),
    )(wm1p, bm1r, wm2, bm2r)


def _bounds_body(batch_ref, starts_ref, ends_ref):
    gv = lax.broadcasted_iota(jnp.int32, (1, G), 1)
    acc_lt = jnp.zeros((1, G), jnp.int32)
    acc_le = jnp.zeros((1, G), jnp.int32)
    for c in range(16):
        bc = batch_ref[:, pl.ds(c, 1)]
        acc_lt = acc_lt + jnp.sum((bc < gv).astype(jnp.int32), axis=0,
                                  keepdims=True)
        acc_le = acc_le + jnp.sum((bc <= gv).astype(jnp.int32), axis=0,
                                  keepdims=True)
    starts_ref[...] = acc_lt
    ends_ref[...] = acc_le


def _make_bounds(batch_t):
    return pl.pallas_call(
        _bounds_body,
        out_shape=(jax.ShapeDtypeStruct((1, G), jnp.int32),
                   jax.ShapeDtypeStruct((1, G), jnp.int32)),
    )(batch_t)


def _h0_body(z_ref, emb_ref, h_ref):
    zt = z_ref[...]
    iot = lax.broadcasted_iota(jnp.int32, (1, 128), 1)
    oneh = (zt == iot).astype(jnp.float32)
    h_ref[...] = jnp.dot(oneh, emb_ref[...],
                         preferred_element_type=jnp.float32)


def _make_h0(z2d, embp):
    return pl.pallas_call(
        _h0_body,
        grid=(N // RT,),
        in_specs=[
            pl.BlockSpec((RT, 1), lambda i: (i, 0)),
            pl.BlockSpec((128, 128), lambda i: (0, 0)),
        ],
        out_specs=pl.BlockSpec((RT, 128), lambda i: (i, 0)),
        out_shape=jax.ShapeDtypeStruct((N, 128), jnp.float32),
    )(z2d, embp)


def _kx_body(h_ref, w_ref, xl_ref):
    xl_ref[...] = jnp.dot(h_ref[...], w_ref[...],
                          preferred_element_type=jnp.float32)


def _make_xl(h, w):
    return pl.pallas_call(
        _kx_body,
        grid=(N // RT,),
        in_specs=[
            pl.BlockSpec((RT, H), lambda i: (i, 0)),
            pl.BlockSpec((H, NF), lambda i: (0, 0)),
        ],
        out_specs=pl.BlockSpec((RT, NF), lambda i: (i, 0)),
        out_shape=jax.ShapeDtypeStruct((N, NF), jnp.float32),
    )(h, w)


def _kh_body(agg_ref, h_ref, wc2_ref, bc2_ref, wl_ref, bl_ref, out_ref):
    x2 = jnp.dot(agg_ref[...], wc2_ref[...],
                 preferred_element_type=jnp.float32) + bc2_ref[...]
    x2 = _ssp(x2)
    x2 = jnp.dot(x2, wl_ref[...], preferred_element_type=jnp.float32) \
        + bl_ref[...]
    out_ref[...] = h_ref[...] + x2


def _make_h_update(agg, h, wc2, bc2r, wl, blr):
    return pl.pallas_call(
        _kh_body,
        grid=(N // RT,),
        in_specs=[
            pl.BlockSpec((RT, NF), lambda i: (i, 0)),
            pl.BlockSpec((RT, H), lambda i: (i, 0)),
            pl.BlockSpec((NF, H), lambda i: (0, 0)),
            pl.BlockSpec((1, H), lambda i: (0, 0)),
            pl.BlockSpec((H, H), lambda i: (0, 0)),
            pl.BlockSpec((1, H), lambda i: (0, 0)),
        ],
        out_specs=pl.BlockSpec((RT, H), lambda i: (i, 0)),
        out_shape=jax.ShapeDtypeStruct((N, H), jnp.float32),
    )(agg, h, wc2, bc2r, wl, blr)


def _kread_body(h_ref, w1_ref, b1_ref, w2_ref, b2_ref, y_ref):
    t = _ssp(jnp.dot(h_ref[...], w1_ref[...],
                     preferred_element_type=jnp.float32) + b1_ref[...])
    y_ref[...] = jnp.dot(t, w2_ref[...],
                         preferred_element_type=jnp.float32) + b2_ref[...]


def _make_read(h, w1, b1r, w2, b2r):
    return pl.pallas_call(
        _kread_body,
        grid=(N // RT,),
        in_specs=[
            pl.BlockSpec((RT, H), lambda i: (i, 0)),
            pl.BlockSpec((H, H // 2), lambda i: (0, 0)),
            pl.BlockSpec((1, H // 2), lambda i: (0, 0)),
            pl.BlockSpec((H // 2, OUT), lambda i: (0, 0)),
            pl.BlockSpec((1, OUT), lambda i: (0, 0)),
        ],
        out_specs=pl.BlockSpec((RT, OUT), lambda i: (i, 0)),
        out_shape=jax.ShapeDtypeStruct((N, OUT), jnp.float32),
    )(h, w1, b1r, w2, b2r)


def _sext(buf, i):
    return buf[pl.ds(i, 16)][0]


def _msg_body(starts_hbm, ends_hbm, px_hbm, py_hbm, pz_hbm, xl_hbm, tab_hbm,
              agg_hbm, tab_v, xl_v, agg_v, pxu, pyu, pzu, pxv, pyv, pzv,
              sv, ev):
    wid = lax.axis_index("s") * NC + lax.axis_index("c")
    base = wid * GPT
    pltpu.sync_copy(tab_hbm, tab_v)
    pltpu.sync_copy(starts_hbm.at[pl.ds(base, GPT)], sv.at[pl.ds(0, GPT)])
    pltpu.sync_copy(ends_hbm.at[pl.ds(base, GPT)], ev.at[pl.ds(0, GPT)])
    lanes = lax.iota(jnp.int32, 16)
    zero16 = jnp.zeros((16,), jnp.float32)

    def graph_body(gl, _):
        s = _sext(sv, gl)
        e = _sext(ev, gl)
        n = e - s

        def vc_body(vc, _):
            v0 = s + vc * CV
            nv = jnp.minimum(CV, n - vc * CV)
            av0 = (v0 // 8) * 8
            voff = v0 - av0
            pltpu.sync_copy(px_hbm.at[pl.ds(av0, CV + 8)],
                            pxv.at[pl.ds(0, CV + 8)])
            pltpu.sync_copy(py_hbm.at[pl.ds(av0, CV + 8)],
                            pyv.at[pl.ds(0, CV + 8)])
            pltpu.sync_copy(pz_hbm.at[pl.ds(av0, CV + 8)],
                            pzv.at[pl.ds(0, CV + 8)])

            def z_body(ivz, _):
                for c2 in range(8):
                    agg_v[pl.ds(ivz * NF + 16 * c2, 16)] = zero16
                return 0

            lax.fori_loop(0, CV, z_body, 0)

            def uc_body(uc, _):
                u0 = s + uc * CU
                nu = jnp.minimum(CU, n - uc * CU)
                au0 = (u0 // 8) * 8
                uoff = u0 - au0
                pltpu.sync_copy(px_hbm.at[pl.ds(au0, CU + 8)],
                                pxu.at[pl.ds(0, CU + 8)])
                pltpu.sync_copy(py_hbm.at[pl.ds(au0, CU + 8)],
                                pyu.at[pl.ds(0, CU + 8)])
                pltpu.sync_copy(pz_hbm.at[pl.ds(au0, CU + 8)],
                                pzu.at[pl.ds(0, CU + 8)])
                pltpu.sync_copy(xl_hbm.at[pl.ds(u0 * NF, CU * NF)], xl_v)
                nut = (nu + 15) // 16

                def v_body(iv, _):
                    v = v0 + iv
                    vx = jnp.full((16,), _sext(pxv, iv + voff))
                    vy = jnp.full((16,), _sext(pyv, iv + voff))
                    vz = jnp.full((16,), _sext(pzv, iv + voff))
                    accs = tuple(agg_v[pl.ds(iv * NF + 16 * c2, 16)]
                                 for c2 in range(8))

                    def ut_body(ut, accs):
                        lane0 = 16 * ut
                        px = pxu[pl.ds(lane0 + uoff, 16)]
                        py = pyu[pl.ds(lane0 + uoff, 16)]
                        pz = pzu[pl.ds(lane0 + uoff, 16)]
                        dx = px - vx
                        dy = py - vy
                        dz = pz - vz
                        d2 = dx * dx + dy * dy + dz * dz
                        ul = lane0 + lanes
                        lane_ok = ul < nu
                        d2 = jnp.where(lane_ok, d2, zero16)
                        sel = lane_ok & (d2 <= CUTOFF * CUTOFF) \
                            & ((u0 + ul) != v)
                        x = jnp.maximum(d2, 1e-24)
                        bits = lax.bitcast_convert_type(x, jnp.int32)
                        bits = 0x1FBD1DF5 + (bits >> 1)
                        y = lax.bitcast_convert_type(bits, jnp.float32)
                        y = 0.5 * (y + x / y)
                        y = 0.5 * (y + x / y)
                        kf = jnp.clip(y * ((K - 1) / CUTOFF) + 0.5,
                                      0.0, float(K - 1))
                        ki = kf.astype(jnp.int32) * NF
                        a16 = jnp.where(sel, 1.0, 0.0).astype(jnp.float32)
                        acc_l = list(accs)
                        for j in range(16):
                            kj = ki[j]
                            wv = jnp.full((16,), a16[j])
                            xbase = (lane0 + j) * NF
                            for c2 in range(8):
                                tvec = tab_v[pl.ds(kj + 16 * c2, 16)]
                                xvec = xl_v[pl.ds(xbase + 16 * c2, 16)]
                                acc_l[c2] = acc_l[c2] + tvec * xvec * wv
                        return tuple(acc_l)

                    accs = lax.fori_loop(0, nut, ut_body, accs)
                    for c2 in range(8):
                        agg_v[pl.ds(iv * NF + 16 * c2, 16)] = accs[c2]
                    return 0

                lax.fori_loop(0, nv, v_body, 0)
                return 0

            nuc = (n + CU - 1) // CU
            lax.fori_loop(0, nuc, uc_body, 0)

            def w_body(iv, _):
                pltpu.sync_copy(agg_v.at[pl.ds(iv * NF, NF)],
                                agg_hbm.at[pl.ds((v0 + iv) * NF, NF)])
                return 0

            lax.fori_loop(0, nv, w_body, 0)
            return 0

        nvc = (n + CV - 1) // CV
        lax.fori_loop(0, nvc, vc_body, 0)
        return 0

    lax.fori_loop(0, GPT, graph_body, 0)


def _make_msg(starts, ends, px, py, pz, xl_p, tab):
    mesh = plsc.VectorSubcoreMesh(core_axis_name="c", subcore_axis_name="s",
                                  num_cores=NC, num_subcores=NS)
    f = pl.kernel(
        _msg_body,
        out_type=jax.ShapeDtypeStruct((N * NF,), jnp.float32),
        mesh=mesh,
        scratch_types=[
            pltpu.VMEM((K * NF,), jnp.float32),
            pltpu.VMEM((CU * NF,), jnp.float32),
            pltpu.VMEM((CV * NF,), jnp.float32),
            pltpu.VMEM((64,), jnp.float32),
            pltpu.VMEM((64,), jnp.float32),
            pltpu.VMEM((64,), jnp.float32),
            pltpu.VMEM((64,), jnp.float32),
            pltpu.VMEM((64,), jnp.float32),
            pltpu.VMEM((64,), jnp.float32),
            pltpu.VMEM((GPT + 16,), jnp.int32),
            pltpu.VMEM((GPT + 16,), jnp.int32),
        ],
    )
    return f(starts, ends, px, py, pz, xl_p, tab).reshape(N, NF)


def _readout_body(starts_hbm, ends_hbm, y_hbm, out_hbm, y_v, row_v, sv, ev):
    wid = lax.axis_index("s") * NC + lax.axis_index("c")
    base = wid * GPT
    pltpu.sync_copy(starts_hbm.at[pl.ds(base, GPT)], sv.at[pl.ds(0, GPT)])
    pltpu.sync_copy(ends_hbm.at[pl.ds(base, GPT)], ev.at[pl.ds(0, GPT)])
    zero16 = jnp.zeros((16,), jnp.float32)

    def graph_body(gl, _):
        s = _sext(sv, gl)
        e = _sext(ev, gl)
        n = e - s
        nuc = (n + CU - 1) // CU

        def uc_body(uc, accs):
            u0 = s + uc * CU
            nu = jnp.minimum(CU, n - uc * CU)
            pltpu.sync_copy(y_hbm.at[pl.ds(u0 * OUT, CU * OUT)], y_v)

            def r_body(r, accs):
                return tuple(accs[c2] + y_v[pl.ds(r * OUT + 16 * c2, 16)]
                             for c2 in range(8))

            return lax.fori_loop(0, nu, r_body, accs)

        accs = lax.fori_loop(0, nuc, uc_body, tuple(zero16 for _ in range(8)))
        for c2 in range(8):
            row_v[pl.ds(16 * c2, 16)] = accs[c2]
        pltpu.sync_copy(row_v, out_hbm.at[pl.ds((base + gl) * OUT, OUT)])
        return 0

    lax.fori_loop(0, GPT, graph_body, 0)


def _make_readout(starts, ends, y_p):
    mesh = plsc.VectorSubcoreMesh(core_axis_name="c", subcore_axis_name="s",
                                  num_cores=NC, num_subcores=NS)
    f = pl.kernel(
        _readout_body,
        out_type=jax.ShapeDtypeStruct((G * OUT,), jnp.float32),
        mesh=mesh,
        scratch_types=[
            pltpu.VMEM((CU * OUT,), jnp.float32),
            pltpu.VMEM((OUT,), jnp.float32),
            pltpu.VMEM((GPT + 16,), jnp.int32),
            pltpu.VMEM((GPT + 16,), jnp.int32),
        ],
    )
    return f(starts, ends, y_p).reshape(G, OUT)


def kernel(z, pos, batch, emb, Wm1, bm1, Wm2, bm2, Wc1, Wc2, bc2, Wl, bl,
           lin1_W, lin1_b, lin2_W, lin2_b):
    z = z.astype(jnp.int32)
    batch = batch.astype(jnp.int32)
    posf = pos.astype(jnp.float32)
    px = jnp.pad(posf[:, 0], (0, 64))
    py = jnp.pad(posf[:, 1], (0, 64))
    pz = jnp.pad(posf[:, 2], (0, 64))
    batch_t = batch.reshape(16, 512).T
    z2d = z.reshape(N, 1)
    embp = jnp.pad(emb, ((0, 28), (0, 0)))
    wm1p = jnp.pad(Wm1, ((0, 0), (0, NGP - NG), (0, 0)))
    bm1r = bm1.reshape(NI, 1, NF)
    bm2r = bm2.reshape(NI, 1, NF)

    tabs = _make_tables(wm1p, bm1r, Wm2, bm2r)
    starts2, ends2 = _make_bounds(batch_t)
    starts = starts2.reshape(G)
    ends = ends2.reshape(G)

    h = _make_h0(z2d, embp)
    for i in range(NI):
        xl = _make_xl(h, Wc1[i])
        xl_p = jnp.pad(xl, ((0, 64), (0, 0))).reshape(-1)
        agg = _make_msg(starts, ends, px, py, pz, xl_p, tabs[i].reshape(-1))
        h = _make_h_update(agg, h, Wc2[i], bc2[i].reshape(1, H), Wl[i],
                           bl[i].reshape(1, H))

    y = _make_read(h, lin1_W, lin1_b.reshape(1, H // 2), lin2_W,
                   lin2_b.reshape(1, OUT))
    y_p = jnp.pad(y, ((0, 64), (0, 0))).reshape(-1)
    return _make_readout(starts, ends, y_p)


# speedup vs baseline: 22.3927x; 1.2442x over previous
import functools

import jax
import jax.numpy as jnp
from jax import lax
from jax.experimental import pallas as pl
from jax.experimental.pallas import tpu as pltpu
from jax.experimental.pallas import tpu_sc as plsc

N = 8192
G = 1024
H = 128
NF = 128
NG = 50
NGP = 64
OUT = 128
NI = 6
CUTOFF = 10.0
K = 512

NC = 2
NS = 16
NW = NC * NS
GPT = G // NW
CU = 32
CV = 32
RT = 256


def _ssp(x):
    return jax.nn.softplus(x) - jnp.log(2.0)


def _tables_body(wm1_ref, bm1_ref, wm2_ref, bm2_ref, t_ref):
    d = lax.broadcasted_iota(jnp.int32, (K, 1), 0).astype(jnp.float32) \
        * (CUTOFF / (K - 1))
    off = lax.broadcasted_iota(jnp.int32, (1, NGP), 1).astype(jnp.float32) \
        * (CUTOFF / (NG - 1))
    step = CUTOFF / (NG - 1)
    coeff = -0.5 / (step * step)
    ea = jnp.exp(coeff * (d - off) ** 2)
    t = _ssp(jnp.dot(ea, wm1_ref[0], preferred_element_type=jnp.float32)
             + bm1_ref[0])
    t = jnp.dot(t, wm2_ref[0], preferred_element_type=jnp.float32) + bm2_ref[0]
    c = 0.5 * (jnp.cos(d * (jnp.pi / CUTOFF)) + 1.0)
    t_ref[0] = t * c


def _make_tables(wm1p, bm1r, wm2, bm2r):
    return pl.pallas_call(
        _tables_body,
        grid=(NI,),
        in_specs=[
            pl.BlockSpec((1, NGP, NF), lambda i: (i, 0, 0)),
            pl.BlockSpec((1, 1, NF), lambda i: (i, 0, 0)),
            pl.BlockSpec((1, NF, NF), lambda i: (i, 0, 0)),
            pl.BlockSpec((1, 1, NF), lambda i: (i, 0, 0)),
        ],
        out_specs=pl.BlockSpec((1, K, NF), lambda i: (i, 0, 0)),
        out_shape=jax.ShapeDtypeStruct((NI, K, NF), jnp.float32),
    )(wm1p, bm1r, wm2, bm2r)


def _bounds_body(batch_ref, starts_ref, ends_ref):
    gv = lax.broadcasted_iota(jnp.int32, (1, G), 1)
    acc_lt = jnp.zeros((1, G), jnp.int32)
    acc_le = jnp.zeros((1, G), jnp.int32)
    for c in range(16):
        bc = batch_ref[:, pl.ds(c, 1)]
        acc_lt = acc_lt + jnp.sum((bc < gv).astype(jnp.int32), axis=0,
                                  keepdims=True)
        acc_le = acc_le + jnp.sum((bc <= gv).astype(jnp.int32), axis=0,
                                  keepdims=True)
    starts_ref[...] = acc_lt
    ends_ref[...] = acc_le


def _make_bounds(batch_t):
    return pl.pallas_call(
        _bounds_body,
        out_shape=(jax.ShapeDtypeStruct((1, G), jnp.int32),
                   jax.ShapeDtypeStruct((1, G), jnp.int32)),
    )(batch_t)


def _h0_body(z_ref, emb_ref, h_ref):
    zt = z_ref[...]
    iot = lax.broadcasted_iota(jnp.int32, (1, 128), 1)
    oneh = (zt == iot).astype(jnp.float32)
    h_ref[...] = jnp.dot(oneh, emb_ref[...],
                         preferred_element_type=jnp.float32)


def _make_h0(z2d, embp):
    return pl.pallas_call(
        _h0_body,
        grid=(N // RT,),
        in_specs=[
            pl.BlockSpec((RT, 1), lambda i: (i, 0)),
            pl.BlockSpec((128, 128), lambda i: (0, 0)),
        ],
        out_specs=pl.BlockSpec((RT, 128), lambda i: (i, 0)),
        out_shape=jax.ShapeDtypeStruct((N, 128), jnp.float32),
    )(z2d, embp)


def _kx_body(h_ref, w_ref, xl_ref):
    xl_ref[...] = jnp.dot(h_ref[...], w_ref[...],
                          preferred_element_type=jnp.float32)


def _make_xl(h, w):
    return pl.pallas_call(
        _kx_body,
        grid=(N // RT,),
        in_specs=[
            pl.BlockSpec((RT, H), lambda i: (i, 0)),
            pl.BlockSpec((H, NF), lambda i: (0, 0)),
        ],
        out_specs=pl.BlockSpec((RT, NF), lambda i: (i, 0)),
        out_shape=jax.ShapeDtypeStruct((N, NF), jnp.float32),
    )(h, w)


def _kh_body(agg_ref, h_ref, wc2_ref, bc2_ref, wl_ref, bl_ref, out_ref):
    x2 = jnp.dot(agg_ref[...], wc2_ref[...],
                 preferred_element_type=jnp.float32) + bc2_ref[...]
    x2 = _ssp(x2)
    x2 = jnp.dot(x2, wl_ref[...], preferred_element_type=jnp.float32) \
        + bl_ref[...]
    out_ref[...] = h_ref[...] + x2


def _make_h_update(agg, h, wc2, bc2r, wl, blr):
    return pl.pallas_call(
        _kh_body,
        grid=(N // RT,),
        in_specs=[
            pl.BlockSpec((RT, NF), lambda i: (i, 0)),
            pl.BlockSpec((RT, H), lambda i: (i, 0)),
            pl.BlockSpec((NF, H), lambda i: (0, 0)),
            pl.BlockSpec((1, H), lambda i: (0, 0)),
            pl.BlockSpec((H, H), lambda i: (0, 0)),
            pl.BlockSpec((1, H), lambda i: (0, 0)),
        ],
        out_specs=pl.BlockSpec((RT, H), lambda i: (i, 0)),
        out_shape=jax.ShapeDtypeStruct((N, H), jnp.float32),
    )(agg, h, wc2, bc2r, wl, blr)


def _kread_body(h_ref, w1_ref, b1_ref, w2_ref, b2_ref, y_ref):
    t = _ssp(jnp.dot(h_ref[...], w1_ref[...],
                     preferred_element_type=jnp.float32) + b1_ref[...])
    y_ref[...] = jnp.dot(t, w2_ref[...],
                         preferred_element_type=jnp.float32) + b2_ref[...]


def _make_read(h, w1, b1r, w2, b2r):
    return pl.pallas_call(
        _kread_body,
        grid=(N // RT,),
        in_specs=[
            pl.BlockSpec((RT, H), lambda i: (i, 0)),
            pl.BlockSpec((H, H // 2), lambda i: (0, 0)),
            pl.BlockSpec((1, H // 2), lambda i: (0, 0)),
            pl.BlockSpec((H // 2, OUT), lambda i: (0, 0)),
            pl.BlockSpec((1, OUT), lambda i: (0, 0)),
        ],
        out_specs=pl.BlockSpec((RT, OUT), lambda i: (i, 0)),
        out_shape=jax.ShapeDtypeStruct((N, OUT), jnp.float32),
    )(h, w1, b1r, w2, b2r)


def _sext(buf, i):
    return buf[pl.ds(i, 16)][0]


def _msg_body(starts_hbm, ends_hbm, px_hbm, py_hbm, pz_hbm, xl_hbm, tab_hbm,
              agg_hbm, tab_v, xl_v, agg_v, pxu, pyu, pzu, pxv, pyv, pzv,
              sv, ev, sem_s, sem_w):
    wid = lax.axis_index("s") * NC + lax.axis_index("c")
    base = wid * GPT
    pltpu.sync_copy(tab_hbm, tab_v)
    pltpu.sync_copy(starts_hbm.at[pl.ds(base, GPT)], sv.at[pl.ds(0, GPT)])
    pltpu.sync_copy(ends_hbm.at[pl.ds(base, GPT)], ev.at[pl.ds(0, GPT)])
    lanes = lax.iota(jnp.int32, 16)
    zero16 = jnp.zeros((16,), jnp.float32)

    def graph_body(gl, _):
        s = _sext(sv, gl)
        e = _sext(ev, gl)
        n = e - s

        def vc_body(vc, _):
            v0 = s + vc * CV
            nv = jnp.minimum(CV, n - vc * CV)
            av0 = (v0 // 8) * 8
            voff = v0 - av0
            d1 = pltpu.async_copy(px_hbm.at[pl.ds(av0, CV + 8)],
                                  pxv.at[pl.ds(0, CV + 8)], sem_s)
            d2 = pltpu.async_copy(py_hbm.at[pl.ds(av0, CV + 8)],
                                  pyv.at[pl.ds(0, CV + 8)], sem_s)
            d3 = pltpu.async_copy(pz_hbm.at[pl.ds(av0, CV + 8)],
                                  pzv.at[pl.ds(0, CV + 8)], sem_s)
            d1.wait()
            d2.wait()
            d3.wait()

            def z_body(ivz, _):
                for c2 in range(8):
                    agg_v[pl.ds(ivz * NF + 16 * c2, 16)] = zero16
                return 0

            lax.fori_loop(0, nv, z_body, 0)

            def uc_body(uc, _):
                u0 = s + uc * CU
                nu = jnp.minimum(CU, n - uc * CU)
                au0 = (u0 // 8) * 8
                uoff = u0 - au0
                e1 = pltpu.async_copy(px_hbm.at[pl.ds(au0, CU + 8)],
                                      pxu.at[pl.ds(0, CU + 8)], sem_s)
                e2 = pltpu.async_copy(py_hbm.at[pl.ds(au0, CU + 8)],
                                      pyu.at[pl.ds(0, CU + 8)], sem_s)
                e3 = pltpu.async_copy(pz_hbm.at[pl.ds(au0, CU + 8)],
                                      pzu.at[pl.ds(0, CU + 8)], sem_s)
                e4 = pltpu.async_copy(xl_hbm.at[pl.ds(u0 * NF, CU * NF)],
                                      xl_v, sem_s)
                e1.wait()
                e2.wait()
                e3.wait()
                e4.wait()
                nut = (nu + 15) // 16

                def v_body(iv):
                    v = v0 + iv
                    vx = jnp.full((16,), _sext(pxv, iv + voff))
                    vy = jnp.full((16,), _sext(pyv, iv + voff))
                    vz = jnp.full((16,), _sext(pzv, iv + voff))
                    accs = tuple(agg_v[pl.ds(iv * NF + 16 * c2, 16)]
                                 for c2 in range(8))

                    def ut_body(ut, accs):
                        lane0 = 16 * ut
                        px = pxu[pl.ds(lane0 + uoff, 16)]
                        py = pyu[pl.ds(lane0 + uoff, 16)]
                        pz = pzu[pl.ds(lane0 + uoff, 16)]
                        dx = px - vx
                        dy = py - vy
                        dz = pz - vz
                        d2 = dx * dx + dy * dy + dz * dz
                        ul = lane0 + lanes
                        lane_ok = ul < nu
                        d2 = jnp.where(lane_ok, d2, zero16)
                        sel = lane_ok & (d2 <= CUTOFF * CUTOFF) \
                            & ((u0 + ul) != v)
                        x = jnp.maximum(d2, 1e-24)
                        bits = lax.bitcast_convert_type(x, jnp.int32)
                        bits = 0x1FBD1DF5 + (bits >> 1)
                        y = lax.bitcast_convert_type(bits, jnp.float32)
                        y = 0.5 * (y + x / y)
                        y = 0.5 * (y + x / y)
                        kf = jnp.clip(y * ((K - 1) / CUTOFF) + 0.5,
                                      0.0, float(K - 1))
                        ki = kf.astype(jnp.int32) * NF
                        a16 = jnp.where(sel, 1.0, 0.0).astype(jnp.float32)
                        acc_l = list(accs)
                        for j in range(16):
                            kj = ki[j]
                            wv = jnp.full((16,), a16[j])
                            xbase = (lane0 + j) * NF
                            for c2 in range(8):
                                tvec = tab_v[pl.ds(kj + 16 * c2, 16)]
                                xvec = xl_v[pl.ds(xbase + 16 * c2, 16)]
                                acc_l[c2] = acc_l[c2] + tvec * xvec * wv
                        return tuple(acc_l)

                    accs = lax.fori_loop(0, nut, ut_body, accs)
                    for c2 in range(8):
                        agg_v[pl.ds(iv * NF + 16 * c2, 16)] = accs[c2]

                plsc.parallel_loop(0, nv, unroll=2)(v_body)
                return 0

            nuc = (n + CU - 1) // CU
            lax.fori_loop(0, nuc, uc_body, 0)

            def w_issue(iv, _):
                pltpu.async_copy(agg_v.at[pl.ds(iv * NF, NF)],
                                 agg_hbm.at[pl.ds((v0 + iv) * NF, NF)],
                                 sem_w)
                return 0

            lax.fori_loop(0, nv, w_issue, 0)

            def w_drain(iv, _):
                pltpu.make_async_copy(
                    agg_hbm.at[pl.ds(0, NF)], agg_v.at[pl.ds(0, NF)],
                    sem_w).wait()
                return 0

            lax.fori_loop(0, nv, w_drain, 0)
            return 0

        nvc = (n + CV - 1) // CV
        lax.fori_loop(0, nvc, vc_body, 0)
        return 0

    lax.fori_loop(0, GPT, graph_body, 0)


def _make_msg(starts, ends, px, py, pz, xl_p, tab):
    mesh = plsc.VectorSubcoreMesh(core_axis_name="c", subcore_axis_name="s",
                                  num_cores=NC, num_subcores=NS)
    f = pl.kernel(
        _msg_body,
        out_type=jax.ShapeDtypeStruct((N * NF,), jnp.float32),
        mesh=mesh,
        scratch_types=[
            pltpu.VMEM((K * NF,), jnp.float32),
            pltpu.VMEM((CU * NF,), jnp.float32),
            pltpu.VMEM((CV * NF,), jnp.float32),
            pltpu.VMEM((64,), jnp.float32),
            pltpu.VMEM((64,), jnp.float32),
            pltpu.VMEM((64,), jnp.float32),
            pltpu.VMEM((64,), jnp.float32),
            pltpu.VMEM((64,), jnp.float32),
            pltpu.VMEM((64,), jnp.float32),
            pltpu.VMEM((GPT + 16,), jnp.int32),
            pltpu.VMEM((GPT + 16,), jnp.int32),
            pltpu.SemaphoreType.DMA,
            pltpu.SemaphoreType.DMA,
        ],
    )
    return f(starts, ends, px, py, pz, xl_p, tab).reshape(N, NF)


def _readout_body(starts_hbm, ends_hbm, y_hbm, out_hbm, y_v, row_v, sv, ev):
    wid = lax.axis_index("s") * NC + lax.axis_index("c")
    base = wid * GPT
    pltpu.sync_copy(starts_hbm.at[pl.ds(base, GPT)], sv.at[pl.ds(0, GPT)])
    pltpu.sync_copy(ends_hbm.at[pl.ds(base, GPT)], ev.at[pl.ds(0, GPT)])
    zero16 = jnp.zeros((16,), jnp.float32)

    def graph_body(gl, _):
        s = _sext(sv, gl)
        e = _sext(ev, gl)
        n = e - s
        nuc = (n + CU - 1) // CU

        def uc_body(uc, accs):
            u0 = s + uc * CU
            nu = jnp.minimum(CU, n - uc * CU)
            pltpu.sync_copy(y_hbm.at[pl.ds(u0 * OUT, CU * OUT)], y_v)

            def r_body(r, accs):
                return tuple(accs[c2] + y_v[pl.ds(r * OUT + 16 * c2, 16)]
                             for c2 in range(8))

            return lax.fori_loop(0, nu, r_body, accs)

        accs = lax.fori_loop(0, nuc, uc_body, tuple(zero16 for _ in range(8)))
        for c2 in range(8):
            row_v[pl.ds(16 * c2, 16)] = accs[c2]
        pltpu.sync_copy(row_v, out_hbm.at[pl.ds((base + gl) * OUT, OUT)])
        return 0

    lax.fori_loop(0, GPT, graph_body, 0)


def _make_readout(starts, ends, y_p):
    mesh = plsc.VectorSubcoreMesh(core_axis_name="c", subcore_axis_name="s",
                                  num_cores=NC, num_subcores=NS)
    f = pl.kernel(
        _readout_body,
        out_type=jax.ShapeDtypeStruct((G * OUT,), jnp.float32),
        mesh=mesh,
        scratch_types=[
            pltpu.VMEM((CU * OUT,), jnp.float32),
            pltpu.VMEM((OUT,), jnp.float32),
            pltpu.VMEM((GPT + 16,), jnp.int32),
            pltpu.VMEM((GPT + 16,), jnp.int32),
        ],
    )
    return f(starts, ends, y_p).reshape(G, OUT)


def kernel(z, pos, batch, emb, Wm1, bm1, Wm2, bm2, Wc1, Wc2, bc2, Wl, bl,
           lin1_W, lin1_b, lin2_W, lin2_b):
    z = z.astype(jnp.int32)
    batch = batch.astype(jnp.int32)
    posf = pos.astype(jnp.float32)
    px = jnp.pad(posf[:, 0], (0, 64))
    py = jnp.pad(posf[:, 1], (0, 64))
    pz = jnp.pad(posf[:, 2], (0, 64))
    batch_t = batch.reshape(16, 512).T
    z2d = z.reshape(N, 1)
    embp = jnp.pad(emb, ((0, 28), (0, 0)))
    wm1p = jnp.pad(Wm1, ((0, 0), (0, NGP - NG), (0, 0)))
    bm1r = bm1.reshape(NI, 1, NF)
    bm2r = bm2.reshape(NI, 1, NF)

    tabs = _make_tables(wm1p, bm1r, Wm2, bm2r)
    starts2, ends2 = _make_bounds(batch_t)
    starts = starts2.reshape(G)
    ends = ends2.reshape(G)

    h = _make_h0(z2d, embp)
    for i in range(NI):
        xl = _make_xl(h, Wc1[i])
        xl_p = jnp.pad(xl, ((0, 64), (0, 0))).reshape(-1)
        agg = _make_msg(starts, ends, px, py, pz, xl_p, tabs[i].reshape(-1))
        h = _make_h_update(agg, h, Wc2[i], bc2[i].reshape(1, H), Wl[i],
                           bl[i].reshape(1, H))

    y = _make_read(h, lin1_W, lin1_b.reshape(1, H // 2), lin2_W,
                   lin2_b.reshape(1, OUT))
    y_p = jnp.pad(y, ((0, 64), (0, 0))).reshape(-1)
    return _make_readout(starts, ends, y_p)


# speedup vs baseline: 24.4918x; 1.0937x over previous
import functools

import jax
import jax.numpy as jnp
from jax import lax
from jax.experimental import pallas as pl
from jax.experimental.pallas import tpu as pltpu
from jax.experimental.pallas import tpu_sc as plsc

N = 8192
G = 1024
H = 128
NF = 128
NG = 50
NGP = 64
OUT = 128
NI = 6
CUTOFF = 10.0
K = 512

NC = 2
NS = 16
NW = NC * NS
GPT = G // NW
CU = 32
CV = 32
RT = 256


def _ssp(x):
    return jax.nn.softplus(x) - jnp.log(2.0)


def _tables_body(wm1_ref, bm1_ref, wm2_ref, bm2_ref, t_ref):
    ki = lax.broadcasted_iota(jnp.int32, (K + 1, 1), 0)
    d = ki.astype(jnp.float32) * (CUTOFF / (K - 1))
    off = lax.broadcasted_iota(jnp.int32, (1, NGP), 1).astype(jnp.float32) \
        * (CUTOFF / (NG - 1))
    step = CUTOFF / (NG - 1)
    coeff = -0.5 / (step * step)
    ea = jnp.exp(coeff * (d - off) ** 2)
    t = _ssp(jnp.dot(ea, wm1_ref[0], preferred_element_type=jnp.float32)
             + bm1_ref[0])
    t = jnp.dot(t, wm2_ref[0], preferred_element_type=jnp.float32) + bm2_ref[0]
    c = 0.5 * (jnp.cos(d * (jnp.pi / CUTOFF)) + 1.0)
    c = jnp.where(ki < K, c, 0.0)
    t_ref[0] = t * c


def _make_tables(wm1p, bm1r, wm2, bm2r):
    return pl.pallas_call(
        _tables_body,
        grid=(NI,),
        in_specs=[
            pl.BlockSpec((1, NGP, NF), lambda i: (i, 0, 0)),
            pl.BlockSpec((1, 1, NF), lambda i: (i, 0, 0)),
            pl.BlockSpec((1, NF, NF), lambda i: (i, 0, 0)),
            pl.BlockSpec((1, 1, NF), lambda i: (i, 0, 0)),
        ],
        out_specs=pl.BlockSpec((1, K + 1, NF), lambda i: (i, 0, 0)),
        out_shape=jax.ShapeDtypeStruct((NI, K + 1, NF), jnp.float32),
    )(wm1p, bm1r, wm2, bm2r)


def _bounds_body(batch_ref, starts_ref, ends_ref):
    gv = lax.broadcasted_iota(jnp.int32, (1, G), 1)
    acc_lt = jnp.zeros((1, G), jnp.int32)
    acc_le = jnp.zeros((1, G), jnp.int32)
    for c in range(16):
        bc = batch_ref[:, pl.ds(c, 1)]
        acc_lt = acc_lt + jnp.sum((bc < gv).astype(jnp.int32), axis=0,
                                  keepdims=True)
        acc_le = acc_le + jnp.sum((bc <= gv).astype(jnp.int32), axis=0,
                                  keepdims=True)
    starts_ref[...] = acc_lt
    ends_ref[...] = acc_le


def _make_bounds(batch_t):
    return pl.pallas_call(
        _bounds_body,
        out_shape=(jax.ShapeDtypeStruct((1, G), jnp.int32),
                   jax.ShapeDtypeStruct((1, G), jnp.int32)),
    )(batch_t)


def _h0_body(z_ref, emb_ref, h_ref):
    zt = z_ref[...]
    iot = lax.broadcasted_iota(jnp.int32, (1, 128), 1)
    oneh = (zt == iot).astype(jnp.float32)
    h_ref[...] = jnp.dot(oneh, emb_ref[...],
                         preferred_element_type=jnp.float32)


def _make_h0(z2d, embp):
    return pl.pallas_call(
        _h0_body,
        grid=(N // RT,),
        in_specs=[
            pl.BlockSpec((RT, 1), lambda i: (i, 0)),
            pl.BlockSpec((128, 128), lambda i: (0, 0)),
        ],
        out_specs=pl.BlockSpec((RT, 128), lambda i: (i, 0)),
        out_shape=jax.ShapeDtypeStruct((N, 128), jnp.float32),
    )(z2d, embp)


def _kx_body(h_ref, w_ref, xl_ref):
    xl_ref[...] = jnp.dot(h_ref[...], w_ref[...],
                          preferred_element_type=jnp.float32)


def _make_xl(h, w):
    return pl.pallas_call(
        _kx_body,
        grid=(N // RT,),
        in_specs=[
            pl.BlockSpec((RT, H), lambda i: (i, 0)),
            pl.BlockSpec((H, NF), lambda i: (0, 0)),
        ],
        out_specs=pl.BlockSpec((RT, NF), lambda i: (i, 0)),
        out_shape=jax.ShapeDtypeStruct((N, NF), jnp.float32),
    )(h, w)


def _kh_body(agg_ref, h_ref, wc2_ref, bc2_ref, wl_ref, bl_ref, out_ref):
    x2 = jnp.dot(agg_ref[...], wc2_ref[...],
                 preferred_element_type=jnp.float32) + bc2_ref[...]
    x2 = _ssp(x2)
    x2 = jnp.dot(x2, wl_ref[...], preferred_element_type=jnp.float32) \
        + bl_ref[...]
    out_ref[...] = h_ref[...] + x2


def _make_h_update(agg, h, wc2, bc2r, wl, blr):
    return pl.pallas_call(
        _kh_body,
        grid=(N // RT,),
        in_specs=[
            pl.BlockSpec((RT, NF), lambda i: (i, 0)),
            pl.BlockSpec((RT, H), lambda i: (i, 0)),
            pl.BlockSpec((NF, H), lambda i: (0, 0)),
            pl.BlockSpec((1, H), lambda i: (0, 0)),
            pl.BlockSpec((H, H), lambda i: (0, 0)),
            pl.BlockSpec((1, H), lambda i: (0, 0)),
        ],
        out_specs=pl.BlockSpec((RT, H), lambda i: (i, 0)),
        out_shape=jax.ShapeDtypeStruct((N, H), jnp.float32),
    )(agg, h, wc2, bc2r, wl, blr)


def _kread_body(h_ref, w1_ref, b1_ref, w2_ref, b2_ref, y_ref):
    t = _ssp(jnp.dot(h_ref[...], w1_ref[...],
                     preferred_element_type=jnp.float32) + b1_ref[...])
    y_ref[...] = jnp.dot(t, w2_ref[...],
                         preferred_element_type=jnp.float32) + b2_ref[...]


def _make_read(h, w1, b1r, w2, b2r):
    return pl.pallas_call(
        _kread_body,
        grid=(N // RT,),
        in_specs=[
            pl.BlockSpec((RT, H), lambda i: (i, 0)),
            pl.BlockSpec((H, H // 2), lambda i: (0, 0)),
            pl.BlockSpec((1, H // 2), lambda i: (0, 0)),
            pl.BlockSpec((H // 2, OUT), lambda i: (0, 0)),
            pl.BlockSpec((1, OUT), lambda i: (0, 0)),
        ],
        out_specs=pl.BlockSpec((RT, OUT), lambda i: (i, 0)),
        out_shape=jax.ShapeDtypeStruct((N, OUT), jnp.float32),
    )(h, w1, b1r, w2, b2r)


def _sext(buf, i):
    return buf[pl.ds(i, 16)][0]


def _msg_body(starts_hbm, ends_hbm, px_hbm, py_hbm, pz_hbm, xl_hbm, tab_hbm,
              agg_hbm, tab_v, xl_v, agg_v, pxu, pyu, pzu, pxv, pyv, pzv,
              sv, ev, sem_s, sem_w):
    wid = lax.axis_index("s") * NC + lax.axis_index("c")
    base = wid * GPT
    pltpu.sync_copy(tab_hbm, tab_v)
    pltpu.sync_copy(starts_hbm.at[pl.ds(base, GPT)], sv.at[pl.ds(0, GPT)])
    pltpu.sync_copy(ends_hbm.at[pl.ds(base, GPT)], ev.at[pl.ds(0, GPT)])
    lanes = lax.iota(jnp.int32, 16)
    zero16 = jnp.zeros((16,), jnp.float32)

    def graph_body(gl, _):
        s = _sext(sv, gl)
        e = _sext(ev, gl)
        n = e - s

        def vc_body(vc, _):
            v0 = s + vc * CV
            nv = jnp.minimum(CV, n - vc * CV)
            av0 = (v0 // 8) * 8
            voff = v0 - av0
            d1 = pltpu.async_copy(px_hbm.at[pl.ds(av0, CV + 8)],
                                  pxv.at[pl.ds(0, CV + 8)], sem_s)
            d2 = pltpu.async_copy(py_hbm.at[pl.ds(av0, CV + 8)],
                                  pyv.at[pl.ds(0, CV + 8)], sem_s)
            d3 = pltpu.async_copy(pz_hbm.at[pl.ds(av0, CV + 8)],
                                  pzv.at[pl.ds(0, CV + 8)], sem_s)
            d1.wait()
            d2.wait()
            d3.wait()

            def z_body(ivz, _):
                for c2 in range(8):
                    agg_v[pl.ds(ivz * NF + 16 * c2, 16)] = zero16
                return 0

            lax.fori_loop(0, nv, z_body, 0)

            def uc_body(uc, _):
                u0 = s + uc * CU
                nu = jnp.minimum(CU, n - uc * CU)
                au0 = (u0 // 8) * 8
                uoff = u0 - au0
                e1 = pltpu.async_copy(px_hbm.at[pl.ds(au0, CU + 8)],
                                      pxu.at[pl.ds(0, CU + 8)], sem_s)
                e2 = pltpu.async_copy(py_hbm.at[pl.ds(au0, CU + 8)],
                                      pyu.at[pl.ds(0, CU + 8)], sem_s)
                e3 = pltpu.async_copy(pz_hbm.at[pl.ds(au0, CU + 8)],
                                      pzu.at[pl.ds(0, CU + 8)], sem_s)
                e4 = pltpu.async_copy(xl_hbm.at[pl.ds(u0 * NF, CU * NF)],
                                      xl_v, sem_s)
                e1.wait()
                e2.wait()
                e3.wait()
                e4.wait()
                nut = (nu + 15) // 16

                def v_body(iv):
                    v = v0 + iv
                    vx = jnp.full((16,), _sext(pxv, iv + voff))
                    vy = jnp.full((16,), _sext(pyv, iv + voff))
                    vz = jnp.full((16,), _sext(pzv, iv + voff))
                    accs = tuple(agg_v[pl.ds(iv * NF + 16 * c2, 16)]
                                 for c2 in range(8))

                    def ut_body(ut, accs):
                        lane0 = 16 * ut
                        px = pxu[pl.ds(lane0 + uoff, 16)]
                        py = pyu[pl.ds(lane0 + uoff, 16)]
                        pz = pzu[pl.ds(lane0 + uoff, 16)]
                        dx = px - vx
                        dy = py - vy
                        dz = pz - vz
                        d2 = dx * dx + dy * dy + dz * dz
                        ul = lane0 + lanes
                        lane_ok = ul < nu
                        d2 = jnp.where(lane_ok, d2, zero16)
                        sel = lane_ok & (d2 <= CUTOFF * CUTOFF) \
                            & ((u0 + ul) != v)
                        x = jnp.maximum(d2, 1e-24)
                        bits = lax.bitcast_convert_type(x, jnp.int32)
                        bits = 0x1FBD1DF5 + (bits >> 1)
                        y = lax.bitcast_convert_type(bits, jnp.float32)
                        y = 0.5 * (y + x / y)
                        y = 0.5 * (y + x / y)
                        kf = jnp.clip(y * ((K - 1) / CUTOFF) + 0.5,
                                      0.0, float(K - 1))
                        ki = kf.astype(jnp.int32) * NF
                        ki = jnp.where(sel, ki, K * NF)
                        acc_l = list(accs)
                        for j in range(16):
                            kj = ki[j]
                            xbase = (lane0 + j) * NF
                            for c2 in range(8):
                                tvec = tab_v[pl.ds(kj + 16 * c2, 16)]
                                xvec = xl_v[pl.ds(xbase + 16 * c2, 16)]
                                acc_l[c2] = acc_l[c2] + tvec * xvec
                        return tuple(acc_l)

                    accs = plsc.parallel_loop(0, nut, carry=accs,
                                              unroll=2)(ut_body)
                    for c2 in range(8):
                        agg_v[pl.ds(iv * NF + 16 * c2, 16)] = accs[c2]

                plsc.parallel_loop(0, nv, unroll=2)(v_body)
                return 0

            nuc = (n + CU - 1) // CU
            lax.fori_loop(0, nuc, uc_body, 0)

            def w_issue(iv, _):
                pltpu.async_copy(agg_v.at[pl.ds(iv * NF, NF)],
                                 agg_hbm.at[pl.ds((v0 + iv) * NF, NF)],
                                 sem_w)
                return 0

            lax.fori_loop(0, nv, w_issue, 0)

            def w_drain(iv, _):
                pltpu.make_async_copy(
                    agg_hbm.at[pl.ds(0, NF)], agg_v.at[pl.ds(0, NF)],
                    sem_w).wait()
                return 0

            lax.fori_loop(0, nv, w_drain, 0)
            return 0

        nvc = (n + CV - 1) // CV
        lax.fori_loop(0, nvc, vc_body, 0)
        return 0

    lax.fori_loop(0, GPT, graph_body, 0)


def _make_msg(starts, ends, px, py, pz, xl_p, tab):
    mesh = plsc.VectorSubcoreMesh(core_axis_name="c", subcore_axis_name="s",
                                  num_cores=NC, num_subcores=NS)
    f = pl.kernel(
        _msg_body,
        out_type=jax.ShapeDtypeStruct((N * NF,), jnp.float32),
        mesh=mesh,
        scratch_types=[
            pltpu.VMEM(((K + 1) * NF,), jnp.float32),
            pltpu.VMEM((CU * NF,), jnp.float32),
            pltpu.VMEM((CV * NF,), jnp.float32),
            pltpu.VMEM((64,), jnp.float32),
            pltpu.VMEM((64,), jnp.float32),
            pltpu.VMEM((64,), jnp.float32),
            pltpu.VMEM((64,), jnp.float32),
            pltpu.VMEM((64,), jnp.float32),
            pltpu.VMEM((64,), jnp.float32),
            pltpu.VMEM((GPT + 16,), jnp.int32),
            pltpu.VMEM((GPT + 16,), jnp.int32),
            pltpu.SemaphoreType.DMA,
            pltpu.SemaphoreType.DMA,
        ],
    )
    return f(starts, ends, px, py, pz, xl_p, tab).reshape(N, NF)


def _readout_body(starts_hbm, ends_hbm, y_hbm, out_hbm, y_v, row_v, sv, ev):
    wid = lax.axis_index("s") * NC + lax.axis_index("c")
    base = wid * GPT
    pltpu.sync_copy(starts_hbm.at[pl.ds(base, GPT)], sv.at[pl.ds(0, GPT)])
    pltpu.sync_copy(ends_hbm.at[pl.ds(base, GPT)], ev.at[pl.ds(0, GPT)])
    zero16 = jnp.zeros((16,), jnp.float32)

    def graph_body(gl, _):
        s = _sext(sv, gl)
        e = _sext(ev, gl)
        n = e - s
        nuc = (n + CU - 1) // CU

        def uc_body(uc, accs):
            u0 = s + uc * CU
            nu = jnp.minimum(CU, n - uc * CU)
            pltpu.sync_copy(y_hbm.at[pl.ds(u0 * OUT, CU * OUT)], y_v)

            def r_body(r, accs):
                return tuple(accs[c2] + y_v[pl.ds(r * OUT + 16 * c2, 16)]
                             for c2 in range(8))

            return lax.fori_loop(0, nu, r_body, accs)

        accs = lax.fori_loop(0, nuc, uc_body, tuple(zero16 for _ in range(8)))
        for c2 in range(8):
            row_v[pl.ds(16 * c2, 16)] = accs[c2]
        pltpu.sync_copy(row_v, out_hbm.at[pl.ds((base + gl) * OUT, OUT)])
        return 0

    lax.fori_loop(0, GPT, graph_body, 0)


def _make_readout(starts, ends, y_p):
    mesh = plsc.VectorSubcoreMesh(core_axis_name="c", subcore_axis_name="s",
                                  num_cores=NC, num_subcores=NS)
    f = pl.kernel(
        _readout_body,
        out_type=jax.ShapeDtypeStruct((G * OUT,), jnp.float32),
        mesh=mesh,
        scratch_types=[
            pltpu.VMEM((CU * OUT,), jnp.float32),
            pltpu.VMEM((OUT,), jnp.float32),
            pltpu.VMEM((GPT + 16,), jnp.int32),
            pltpu.VMEM((GPT + 16,), jnp.int32),
        ],
    )
    return f(starts, ends, y_p).reshape(G, OUT)


def kernel(z, pos, batch, emb, Wm1, bm1, Wm2, bm2, Wc1, Wc2, bc2, Wl, bl,
           lin1_W, lin1_b, lin2_W, lin2_b):
    z = z.astype(jnp.int32)
    batch = batch.astype(jnp.int32)
    posf = pos.astype(jnp.float32)
    px = jnp.pad(posf[:, 0], (0, 64))
    py = jnp.pad(posf[:, 1], (0, 64))
    pz = jnp.pad(posf[:, 2], (0, 64))
    batch_t = batch.reshape(16, 512).T
    z2d = z.reshape(N, 1)
    embp = jnp.pad(emb, ((0, 28), (0, 0)))
    wm1p = jnp.pad(Wm1, ((0, 0), (0, NGP - NG), (0, 0)))
    bm1r = bm1.reshape(NI, 1, NF)
    bm2r = bm2.reshape(NI, 1, NF)

    tabs = _make_tables(wm1p, bm1r, Wm2, bm2r)
    starts2, ends2 = _make_bounds(batch_t)
    starts = starts2.reshape(G)
    ends = ends2.reshape(G)

    h = _make_h0(z2d, embp)
    for i in range(NI):
        xl = _make_xl(h, Wc1[i])
        xl_p = jnp.pad(xl, ((0, 64), (0, 0))).reshape(-1)
        agg = _make_msg(starts, ends, px, py, pz, xl_p, tabs[i].reshape(-1))
        h = _make_h_update(agg, h, Wc2[i], bc2[i].reshape(1, H), Wl[i],
                           bl[i].reshape(1, H))

    y = _make_read(h, lin1_W, lin1_b.reshape(1, H // 2), lin2_W,
                   lin2_b.reshape(1, OUT))
    y_p = jnp.pad(y, ((0, 64), (0, 0))).reshape(-1)
    return _make_readout(starts, ends, y_p)


# speedup vs baseline: 26.4350x; 1.0793x over previous
import functools

import jax
import jax.numpy as jnp
from jax import lax
from jax.experimental import pallas as pl
from jax.experimental.pallas import tpu as pltpu
from jax.experimental.pallas import tpu_sc as plsc

N = 8192
G = 1024
H = 128
NF = 128
NG = 50
NGP = 64
OUT = 128
NI = 6
CUTOFF = 10.0
K = 512

NC = 2
NS = 16
NW = NC * NS
GPT = G // NW
CU = 32
CV = 32
RT = 256


def _ssp(x):
    return jax.nn.softplus(x) - jnp.log(2.0)


def _tables_body(wm1_ref, bm1_ref, wm2_ref, bm2_ref, t_ref):
    ki = lax.broadcasted_iota(jnp.int32, (K + 1, 1), 0)
    d = ki.astype(jnp.float32) * (CUTOFF / (K - 1))
    off = lax.broadcasted_iota(jnp.int32, (1, NGP), 1).astype(jnp.float32) \
        * (CUTOFF / (NG - 1))
    step = CUTOFF / (NG - 1)
    coeff = -0.5 / (step * step)
    ea = jnp.exp(coeff * (d - off) ** 2)
    t = _ssp(jnp.dot(ea, wm1_ref[0], preferred_element_type=jnp.float32)
             + bm1_ref[0])
    t = jnp.dot(t, wm2_ref[0], preferred_element_type=jnp.float32) + bm2_ref[0]
    c = 0.5 * (jnp.cos(d * (jnp.pi / CUTOFF)) + 1.0)
    c = jnp.where(ki < K, c, 0.0)
    t_ref[0] = t * c


def _make_tables(wm1p, bm1r, wm2, bm2r):
    return pl.pallas_call(
        _tables_body,
        grid=(NI,),
        in_specs=[
            pl.BlockSpec((1, NGP, NF), lambda i: (i, 0, 0)),
            pl.BlockSpec((1, 1, NF), lambda i: (i, 0, 0)),
            pl.BlockSpec((1, NF, NF), lambda i: (i, 0, 0)),
            pl.BlockSpec((1, 1, NF), lambda i: (i, 0, 0)),
        ],
        out_specs=pl.BlockSpec((1, K + 1, NF), lambda i: (i, 0, 0)),
        out_shape=jax.ShapeDtypeStruct((NI, K + 1, NF), jnp.float32),
    )(wm1p, bm1r, wm2, bm2r)


def _bounds_body(batch_ref, starts_ref, ends_ref):
    gv = lax.broadcasted_iota(jnp.int32, (1, G), 1)
    acc_lt = jnp.zeros((1, G), jnp.int32)
    acc_le = jnp.zeros((1, G), jnp.int32)
    for c in range(16):
        bc = batch_ref[:, pl.ds(c, 1)]
        acc_lt = acc_lt + jnp.sum((bc < gv).astype(jnp.int32), axis=0,
                                  keepdims=True)
        acc_le = acc_le + jnp.sum((bc <= gv).astype(jnp.int32), axis=0,
                                  keepdims=True)
    starts_ref[...] = acc_lt
    ends_ref[...] = acc_le


def _make_bounds(batch_t):
    return pl.pallas_call(
        _bounds_body,
        out_shape=(jax.ShapeDtypeStruct((1, G), jnp.int32),
                   jax.ShapeDtypeStruct((1, G), jnp.int32)),
    )(batch_t)


def _h0x_body(z_ref, emb_ref, wc1_ref, h_ref, xl_ref):
    zt = z_ref[...]
    iot = lax.broadcasted_iota(jnp.int32, (1, 128), 1)
    oneh = (zt == iot).astype(jnp.float32)
    h = jnp.dot(oneh, emb_ref[...], preferred_element_type=jnp.float32)
    h_ref[...] = h
    xl_ref[...] = jnp.dot(h, wc1_ref[...], preferred_element_type=jnp.float32)


def _make_h0_xl(z2d, embp, wc1):
    return pl.pallas_call(
        _h0x_body,
        grid=(N // RT,),
        in_specs=[
            pl.BlockSpec((RT, 1), lambda i: (i, 0)),
            pl.BlockSpec((128, 128), lambda i: (0, 0)),
            pl.BlockSpec((H, NF), lambda i: (0, 0)),
        ],
        out_specs=[
            pl.BlockSpec((RT, 128), lambda i: (i, 0)),
            pl.BlockSpec((RT, NF), lambda i: (i, 0)),
        ],
        out_shape=[jax.ShapeDtypeStruct((N, 128), jnp.float32),
                   jax.ShapeDtypeStruct((N, NF), jnp.float32)],
    )(z2d, embp, wc1)


def _khx_body(agg_ref, h_ref, wc2_ref, bc2_ref, wl_ref, bl_ref, wc1_ref,
              hout_ref, xl_ref):
    x2 = jnp.dot(agg_ref[...], wc2_ref[...],
                 preferred_element_type=jnp.float32) + bc2_ref[...]
    x2 = _ssp(x2)
    x2 = jnp.dot(x2, wl_ref[...], preferred_element_type=jnp.float32) \
        + bl_ref[...]
    h = h_ref[...] + x2
    hout_ref[...] = h
    xl_ref[...] = jnp.dot(h, wc1_ref[...], preferred_element_type=jnp.float32)


def _make_h_update_xl(agg, h, wc2, bc2r, wl, blr, wc1n):
    return pl.pallas_call(
        _khx_body,
        grid=(N // RT,),
        in_specs=[
            pl.BlockSpec((RT, NF), lambda i: (i, 0)),
            pl.BlockSpec((RT, H), lambda i: (i, 0)),
            pl.BlockSpec((NF, H), lambda i: (0, 0)),
            pl.BlockSpec((1, H), lambda i: (0, 0)),
            pl.BlockSpec((H, H), lambda i: (0, 0)),
            pl.BlockSpec((1, H), lambda i: (0, 0)),
            pl.BlockSpec((H, NF), lambda i: (0, 0)),
        ],
        out_specs=[
            pl.BlockSpec((RT, H), lambda i: (i, 0)),
            pl.BlockSpec((RT, NF), lambda i: (i, 0)),
        ],
        out_shape=[jax.ShapeDtypeStruct((N, H), jnp.float32),
                   jax.ShapeDtypeStruct((N, NF), jnp.float32)],
    )(agg, h, wc2, bc2r, wl, blr, wc1n)


def _khread_body(agg_ref, h_ref, wc2_ref, bc2_ref, wl_ref, bl_ref,
                 w1_ref, b1_ref, w2_ref, b2_ref, y_ref):
    x2 = jnp.dot(agg_ref[...], wc2_ref[...],
                 preferred_element_type=jnp.float32) + bc2_ref[...]
    x2 = _ssp(x2)
    x2 = jnp.dot(x2, wl_ref[...], preferred_element_type=jnp.float32) \
        + bl_ref[...]
    h = h_ref[...] + x2
    t = _ssp(jnp.dot(h, w1_ref[...],
                     preferred_element_type=jnp.float32) + b1_ref[...])
    y_ref[...] = jnp.dot(t, w2_ref[...],
                         preferred_element_type=jnp.float32) + b2_ref[...]


def _make_h_update_read(agg, h, wc2, bc2r, wl, blr, w1, b1r, w2, b2r):
    return pl.pallas_call(
        _khread_body,
        grid=(N // RT,),
        in_specs=[
            pl.BlockSpec((RT, NF), lambda i: (i, 0)),
            pl.BlockSpec((RT, H), lambda i: (i, 0)),
            pl.BlockSpec((NF, H), lambda i: (0, 0)),
            pl.BlockSpec((1, H), lambda i: (0, 0)),
            pl.BlockSpec((H, H), lambda i: (0, 0)),
            pl.BlockSpec((1, H), lambda i: (0, 0)),
            pl.BlockSpec((H, H // 2), lambda i: (0, 0)),
            pl.BlockSpec((1, H // 2), lambda i: (0, 0)),
            pl.BlockSpec((H // 2, OUT), lambda i: (0, 0)),
            pl.BlockSpec((1, OUT), lambda i: (0, 0)),
        ],
        out_specs=pl.BlockSpec((RT, OUT), lambda i: (i, 0)),
        out_shape=jax.ShapeDtypeStruct((N, OUT), jnp.float32),
    )(agg, h, wc2, bc2r, wl, blr, w1, b1r, w2, b2r)


def _sext(buf, i):
    return buf[pl.ds(i, 16)][0]


def _msg_body(starts_hbm, ends_hbm, px_hbm, py_hbm, pz_hbm, xl_hbm, tab_hbm,
              agg_hbm, tab_v, xl_v, agg_v, pxu, pyu, pzu, pxv, pyv, pzv,
              sv, ev, sem_s, sem_w):
    wid = lax.axis_index("s") * NC + lax.axis_index("c")
    base = wid * GPT
    pltpu.sync_copy(tab_hbm, tab_v)
    pltpu.sync_copy(starts_hbm.at[pl.ds(base, GPT)], sv.at[pl.ds(0, GPT)])
    pltpu.sync_copy(ends_hbm.at[pl.ds(base, GPT)], ev.at[pl.ds(0, GPT)])
    lanes = lax.iota(jnp.int32, 16)
    zero16 = jnp.zeros((16,), jnp.float32)

    def graph_body(gl, _):
        s = _sext(sv, gl)
        e = _sext(ev, gl)
        n = e - s

        def vc_body(vc, _):
            v0 = s + vc * CV
            nv = jnp.minimum(CV, n - vc * CV)
            av0 = (v0 // 8) * 8
            voff = v0 - av0
            d1 = pltpu.async_copy(px_hbm.at[pl.ds(av0, CV + 8)],
                                  pxv.at[pl.ds(0, CV + 8)], sem_s)
            d2 = pltpu.async_copy(py_hbm.at[pl.ds(av0, CV + 8)],
                                  pyv.at[pl.ds(0, CV + 8)], sem_s)
            d3 = pltpu.async_copy(pz_hbm.at[pl.ds(av0, CV + 8)],
                                  pzv.at[pl.ds(0, CV + 8)], sem_s)
            d1.wait()
            d2.wait()
            d3.wait()

            def z_body(ivz, _):
                for c2 in range(8):
                    agg_v[pl.ds(ivz * NF + 16 * c2, 16)] = zero16
                return 0

            lax.fori_loop(0, nv, z_body, 0)

            def uc_body(uc, _):
                u0 = s + uc * CU
                nu = jnp.minimum(CU, n - uc * CU)
                au0 = (u0 // 8) * 8
                uoff = u0 - au0
                e1 = pltpu.async_copy(px_hbm.at[pl.ds(au0, CU + 8)],
                                      pxu.at[pl.ds(0, CU + 8)], sem_s)
                e2 = pltpu.async_copy(py_hbm.at[pl.ds(au0, CU + 8)],
                                      pyu.at[pl.ds(0, CU + 8)], sem_s)
                e3 = pltpu.async_copy(pz_hbm.at[pl.ds(au0, CU + 8)],
                                      pzu.at[pl.ds(0, CU + 8)], sem_s)
                e4 = pltpu.async_copy(xl_hbm.at[pl.ds(u0 * NF, CU * NF)],
                                      xl_v, sem_s)
                e1.wait()
                e2.wait()
                e3.wait()
                e4.wait()
                nut = (nu + 15) // 16

                def v_body(iv):
                    v = v0 + iv
                    vx = jnp.full((16,), _sext(pxv, iv + voff))
                    vy = jnp.full((16,), _sext(pyv, iv + voff))
                    vz = jnp.full((16,), _sext(pzv, iv + voff))
                    accs = tuple(agg_v[pl.ds(iv * NF + 16 * c2, 16)]
                                 for c2 in range(8))

                    def ut_body(ut, accs):
                        lane0 = 16 * ut
                        px = pxu[pl.ds(lane0 + uoff, 16)]
                        py = pyu[pl.ds(lane0 + uoff, 16)]
                        pz = pzu[pl.ds(lane0 + uoff, 16)]
                        dx = px - vx
                        dy = py - vy
                        dz = pz - vz
                        d2 = dx * dx + dy * dy + dz * dz
                        ul = lane0 + lanes
                        lane_ok = ul < nu
                        d2 = jnp.where(lane_ok, d2, zero16)
                        sel = lane_ok & (d2 <= CUTOFF * CUTOFF) \
                            & ((u0 + ul) != v)
                        x = jnp.maximum(d2, 1e-24)
                        bits = lax.bitcast_convert_type(x, jnp.int32)
                        bits = 0x1FBD1DF5 + (bits >> 1)
                        y = lax.bitcast_convert_type(bits, jnp.float32)
                        y = 0.5 * (y + x / y)
                        y = 0.5 * (y + x / y)
                        kf = jnp.clip(y * ((K - 1) / CUTOFF) + 0.5,
                                      0.0, float(K - 1))
                        ki = kf.astype(jnp.int32) * NF
                        ki = jnp.where(sel, ki, K * NF)
                        acc_l = list(accs)
                        for j in range(16):
                            kj = ki[j]
                            xbase = (lane0 + j) * NF
                            for c2 in range(8):
                                tvec = tab_v[pl.ds(kj + 16 * c2, 16)]
                                xvec = xl_v[pl.ds(xbase + 16 * c2, 16)]
                                acc_l[c2] = acc_l[c2] + tvec * xvec
                        return tuple(acc_l)

                    accs = plsc.parallel_loop(0, nut, carry=accs,
                                              unroll=2)(ut_body)
                    for c2 in range(8):
                        agg_v[pl.ds(iv * NF + 16 * c2, 16)] = accs[c2]

                plsc.parallel_loop(0, nv, unroll=2)(v_body)
                return 0

            nuc = (n + CU - 1) // CU
            lax.fori_loop(0, nuc, uc_body, 0)

            def w_issue(iv, _):
                pltpu.async_copy(agg_v.at[pl.ds(iv * NF, NF)],
                                 agg_hbm.at[pl.ds((v0 + iv) * NF, NF)],
                                 sem_w)
                return 0

            lax.fori_loop(0, nv, w_issue, 0)

            def w_drain(iv, _):
                pltpu.make_async_copy(
                    agg_hbm.at[pl.ds(0, NF)], agg_v.at[pl.ds(0, NF)],
                    sem_w).wait()
                return 0

            lax.fori_loop(0, nv, w_drain, 0)
            return 0

        nvc = (n + CV - 1) // CV
        lax.fori_loop(0, nvc, vc_body, 0)
        return 0

    lax.fori_loop(0, GPT, graph_body, 0)


def _make_msg(starts, ends, px, py, pz, xl_p, tab):
    mesh = plsc.VectorSubcoreMesh(core_axis_name="c", subcore_axis_name="s",
                                  num_cores=NC, num_subcores=NS)
    f = pl.kernel(
        _msg_body,
        out_type=jax.ShapeDtypeStruct((N * NF,), jnp.float32),
        mesh=mesh,
        scratch_types=[
            pltpu.VMEM(((K + 1) * NF,), jnp.float32),
            pltpu.VMEM((CU * NF,), jnp.float32),
            pltpu.VMEM((CV * NF,), jnp.float32),
            pltpu.VMEM((64,), jnp.float32),
            pltpu.VMEM((64,), jnp.float32),
            pltpu.VMEM((64,), jnp.float32),
            pltpu.VMEM((64,), jnp.float32),
            pltpu.VMEM((64,), jnp.float32),
            pltpu.VMEM((64,), jnp.float32),
            pltpu.VMEM((GPT + 16,), jnp.int32),
            pltpu.VMEM((GPT + 16,), jnp.int32),
            pltpu.SemaphoreType.DMA,
            pltpu.SemaphoreType.DMA,
        ],
    )
    return f(starts, ends, px, py, pz, xl_p, tab).reshape(N, NF)


def _readout_body(starts_hbm, ends_hbm, y_hbm, out_hbm, y_v, row_v, sv, ev):
    wid = lax.axis_index("s") * NC + lax.axis_index("c")
    base = wid * GPT
    pltpu.sync_copy(starts_hbm.at[pl.ds(base, GPT)], sv.at[pl.ds(0, GPT)])
    pltpu.sync_copy(ends_hbm.at[pl.ds(base, GPT)], ev.at[pl.ds(0, GPT)])
    zero16 = jnp.zeros((16,), jnp.float32)

    def graph_body(gl, _):
        s = _sext(sv, gl)
        e = _sext(ev, gl)
        n = e - s
        nuc = (n + CU - 1) // CU

        def uc_body(uc, accs):
            u0 = s + uc * CU
            nu = jnp.minimum(CU, n - uc * CU)
            pltpu.sync_copy(y_hbm.at[pl.ds(u0 * OUT, CU * OUT)], y_v)

            def r_body(r, accs):
                return tuple(accs[c2] + y_v[pl.ds(r * OUT + 16 * c2, 16)]
                             for c2 in range(8))

            return lax.fori_loop(0, nu, r_body, accs)

        accs = lax.fori_loop(0, nuc, uc_body, tuple(zero16 for _ in range(8)))
        for c2 in range(8):
            row_v[pl.ds(16 * c2, 16)] = accs[c2]
        pltpu.sync_copy(row_v, out_hbm.at[pl.ds((base + gl) * OUT, OUT)])
        return 0

    lax.fori_loop(0, GPT, graph_body, 0)


def _make_readout(starts, ends, y_p):
    mesh = plsc.VectorSubcoreMesh(core_axis_name="c", subcore_axis_name="s",
                                  num_cores=NC, num_subcores=NS)
    f = pl.kernel(
        _readout_body,
        out_type=jax.ShapeDtypeStruct((G * OUT,), jnp.float32),
        mesh=mesh,
        scratch_types=[
            pltpu.VMEM((CU * OUT,), jnp.float32),
            pltpu.VMEM((OUT,), jnp.float32),
            pltpu.VMEM((GPT + 16,), jnp.int32),
            pltpu.VMEM((GPT + 16,), jnp.int32),
        ],
    )
    return f(starts, ends, y_p).reshape(G, OUT)


def kernel(z, pos, batch, emb, Wm1, bm1, Wm2, bm2, Wc1, Wc2, bc2, Wl, bl,
           lin1_W, lin1_b, lin2_W, lin2_b):
    z = z.astype(jnp.int32)
    batch = batch.astype(jnp.int32)
    posf = pos.astype(jnp.float32)
    px = jnp.pad(posf[:, 0], (0, 64))
    py = jnp.pad(posf[:, 1], (0, 64))
    pz = jnp.pad(posf[:, 2], (0, 64))
    batch_t = batch.reshape(16, 512).T
    z2d = z.reshape(N, 1)
    embp = jnp.pad(emb, ((0, 28), (0, 0)))
    wm1p = jnp.pad(Wm1, ((0, 0), (0, NGP - NG), (0, 0)))
    bm1r = bm1.reshape(NI, 1, NF)
    bm2r = bm2.reshape(NI, 1, NF)

    tabs = _make_tables(wm1p, bm1r, Wm2, bm2r)
    starts2, ends2 = _make_bounds(batch_t)
    starts = starts2.reshape(G)
    ends = ends2.reshape(G)

    h, xl = _make_h0_xl(z2d, embp, Wc1[0])
    for i in range(NI):
        xl_p = jnp.pad(xl, ((0, 64), (0, 0))).reshape(-1)
        agg = _make_msg(starts, ends, px, py, pz, xl_p, tabs[i].reshape(-1))
        if i < NI - 1:
            h, xl = _make_h_update_xl(agg, h, Wc2[i], bc2[i].reshape(1, H),
                                      Wl[i], bl[i].reshape(1, H), Wc1[i + 1])
        else:
            y = _make_h_update_read(agg, h, Wc2[i], bc2[i].reshape(1, H),
                                    Wl[i], bl[i].reshape(1, H),
                                    lin1_W, lin1_b.reshape(1, H // 2),
                                    lin2_W, lin2_b.reshape(1, OUT))
    y_p = jnp.pad(y, ((0, 64), (0, 0))).reshape(-1)
    return _make_readout(starts, ends, y_p)


# speedup vs baseline: 27.4254x; 1.0375x over previous
import functools

import jax
import jax.numpy as jnp
from jax import lax
from jax.experimental import pallas as pl
from jax.experimental.pallas import tpu as pltpu
from jax.experimental.pallas import tpu_sc as plsc

N = 8192
G = 1024
H = 128
NF = 128
NG = 50
NGP = 64
OUT = 128
NI = 6
CUTOFF = 10.0
K = 512

NC = 2
NS = 16
NW = NC * NS
GPT = G // NW
CV = 32
CU = 32
SU = 384
RT = 256
PPAD = SU + 64


def _ssp(x):
    return jax.nn.softplus(x) - jnp.log(2.0)


def _tables_body(wm1_ref, bm1_ref, wm2_ref, bm2_ref, t_ref):
    ki = lax.broadcasted_iota(jnp.int32, (K + 1, 1), 0)
    d = ki.astype(jnp.float32) * (CUTOFF / (K - 1))
    off = lax.broadcasted_iota(jnp.int32, (1, NGP), 1).astype(jnp.float32) \
        * (CUTOFF / (NG - 1))
    step = CUTOFF / (NG - 1)
    coeff = -0.5 / (step * step)
    ea = jnp.exp(coeff * (d - off) ** 2)
    t = _ssp(jnp.dot(ea, wm1_ref[0], preferred_element_type=jnp.float32)
             + bm1_ref[0])
    t = jnp.dot(t, wm2_ref[0], preferred_element_type=jnp.float32) + bm2_ref[0]
    c = 0.5 * (jnp.cos(d * (jnp.pi / CUTOFF)) + 1.0)
    c = jnp.where(ki < K, c, 0.0)
    t_ref[0] = t * c


def _make_tables(wm1p, bm1r, wm2, bm2r):
    return pl.pallas_call(
        _tables_body,
        grid=(NI,),
        in_specs=[
            pl.BlockSpec((1, NGP, NF), lambda i: (i, 0, 0)),
            pl.BlockSpec((1, 1, NF), lambda i: (i, 0, 0)),
            pl.BlockSpec((1, NF, NF), lambda i: (i, 0, 0)),
            pl.BlockSpec((1, 1, NF), lambda i: (i, 0, 0)),
        ],
        out_specs=pl.BlockSpec((1, K + 1, NF), lambda i: (i, 0, 0)),
        out_shape=jax.ShapeDtypeStruct((NI, K + 1, NF), jnp.float32),
    )(wm1p, bm1r, wm2, bm2r)


def _bounds_body(batch_ref, starts_ref, ends_ref):
    gv = lax.broadcasted_iota(jnp.int32, (1, G), 1)
    acc_lt = jnp.zeros((1, G), jnp.int32)
    acc_le = jnp.zeros((1, G), jnp.int32)
    for c in range(16):
        bc = batch_ref[:, pl.ds(c, 1)]
        acc_lt = acc_lt + jnp.sum((bc < gv).astype(jnp.int32), axis=0,
                                  keepdims=True)
        acc_le = acc_le + jnp.sum((bc <= gv).astype(jnp.int32), axis=0,
                                  keepdims=True)
    starts_ref[...] = acc_lt
    ends_ref[...] = acc_le


def _make_bounds(batch_t):
    return pl.pallas_call(
        _bounds_body,
        out_shape=(jax.ShapeDtypeStruct((1, G), jnp.int32),
                   jax.ShapeDtypeStruct((1, G), jnp.int32)),
    )(batch_t)


def _h0x_body(z_ref, emb_ref, wc1_ref, h_ref, xl_ref):
    zt = z_ref[...]
    iot = lax.broadcasted_iota(jnp.int32, (1, 128), 1)
    oneh = (zt == iot).astype(jnp.float32)
    h = jnp.dot(oneh, emb_ref[...], preferred_element_type=jnp.float32)
    h_ref[...] = h
    xl_ref[...] = jnp.dot(h, wc1_ref[...], preferred_element_type=jnp.float32)


def _make_h0_xl(z2d, embp, wc1):
    return pl.pallas_call(
        _h0x_body,
        grid=(N // RT,),
        in_specs=[
            pl.BlockSpec((RT, 1), lambda i: (i, 0)),
            pl.BlockSpec((128, 128), lambda i: (0, 0)),
            pl.BlockSpec((H, NF), lambda i: (0, 0)),
        ],
        out_specs=[
            pl.BlockSpec((RT, 128), lambda i: (i, 0)),
            pl.BlockSpec((RT, NF), lambda i: (i, 0)),
        ],
        out_shape=[jax.ShapeDtypeStruct((N, 128), jnp.float32),
                   jax.ShapeDtypeStruct((N, NF), jnp.float32)],
    )(z2d, embp, wc1)


def _khx_body(agg_ref, h_ref, wc2_ref, bc2_ref, wl_ref, bl_ref, wc1_ref,
              hout_ref, xl_ref):
    x2 = jnp.dot(agg_ref[...], wc2_ref[...],
                 preferred_element_type=jnp.float32) + bc2_ref[...]
    x2 = _ssp(x2)
    x2 = jnp.dot(x2, wl_ref[...], preferred_element_type=jnp.float32) \
        + bl_ref[...]
    h = h_ref[...] + x2
    hout_ref[...] = h
    xl_ref[...] = jnp.dot(h, wc1_ref[...], preferred_element_type=jnp.float32)


def _make_h_update_xl(agg, h, wc2, bc2r, wl, blr, wc1n):
    return pl.pallas_call(
        _khx_body,
        grid=(N // RT,),
        in_specs=[
            pl.BlockSpec((RT, NF), lambda i: (i, 0)),
            pl.BlockSpec((RT, H), lambda i: (i, 0)),
            pl.BlockSpec((NF, H), lambda i: (0, 0)),
            pl.BlockSpec((1, H), lambda i: (0, 0)),
            pl.BlockSpec((H, H), lambda i: (0, 0)),
            pl.BlockSpec((1, H), lambda i: (0, 0)),
            pl.BlockSpec((H, NF), lambda i: (0, 0)),
        ],
        out_specs=[
            pl.BlockSpec((RT, H), lambda i: (i, 0)),
            pl.BlockSpec((RT, NF), lambda i: (i, 0)),
        ],
        out_shape=[jax.ShapeDtypeStruct((N, H), jnp.float32),
                   jax.ShapeDtypeStruct((N, NF), jnp.float32)],
    )(agg, h, wc2, bc2r, wl, blr, wc1n)


def _khread_body(agg_ref, h_ref, wc2_ref, bc2_ref, wl_ref, bl_ref,
                 w1_ref, b1_ref, w2_ref, b2_ref, y_ref):
    x2 = jnp.dot(agg_ref[...], wc2_ref[...],
                 preferred_element_type=jnp.float32) + bc2_ref[...]
    x2 = _ssp(x2)
    x2 = jnp.dot(x2, wl_ref[...], preferred_element_type=jnp.float32) \
        + bl_ref[...]
    h = h_ref[...] + x2
    t = _ssp(jnp.dot(h, w1_ref[...],
                     preferred_element_type=jnp.float32) + b1_ref[...])
    y_ref[...] = jnp.dot(t, w2_ref[...],
                         preferred_element_type=jnp.float32) + b2_ref[...]


def _make_h_update_read(agg, h, wc2, bc2r, wl, blr, w1, b1r, w2, b2r):
    return pl.pallas_call(
        _khread_body,
        grid=(N // RT,),
        in_specs=[
            pl.BlockSpec((RT, NF), lambda i: (i, 0)),
            pl.BlockSpec((RT, H), lambda i: (i, 0)),
            pl.BlockSpec((NF, H), lambda i: (0, 0)),
            pl.BlockSpec((1, H), lambda i: (0, 0)),
            pl.BlockSpec((H, H), lambda i: (0, 0)),
            pl.BlockSpec((1, H), lambda i: (0, 0)),
            pl.BlockSpec((H, H // 2), lambda i: (0, 0)),
            pl.BlockSpec((1, H // 2), lambda i: (0, 0)),
            pl.BlockSpec((H // 2, OUT), lambda i: (0, 0)),
            pl.BlockSpec((1, OUT), lambda i: (0, 0)),
        ],
        out_specs=pl.BlockSpec((RT, OUT), lambda i: (i, 0)),
        out_shape=jax.ShapeDtypeStruct((N, OUT), jnp.float32),
    )(agg, h, wc2, bc2r, wl, blr, w1, b1r, w2, b2r)


def _sext(buf, i):
    return buf[pl.ds(i, 16)][0]


def _msg_body(starts_hbm, ends_hbm, px_hbm, py_hbm, pz_hbm, xl_hbm, tab_hbm,
              agg_hbm, tab_v, xlw, agg_v, pwx, pwy, pwz, pxv, pyv, pzv,
              sv, ev, sem_s, sem_w):
    wid = lax.axis_index("s") * NC + lax.axis_index("c")
    base = wid * GPT
    pltpu.sync_copy(tab_hbm, tab_v)
    pltpu.sync_copy(starts_hbm.at[pl.ds(base, GPT)], sv.at[pl.ds(0, GPT)])
    pltpu.sync_copy(ends_hbm.at[pl.ds(base, GPT)], ev.at[pl.ds(0, GPT)])
    lanes = lax.iota(jnp.int32, 16)
    zero16 = jnp.zeros((16,), jnp.float32)

    w0 = _sext(sv, 0)
    w1 = _sext(ev, GPT - 1)
    wlen = w1 - w0
    whole = wlen <= SU
    aw0 = (w0 // 8) * 8

    @pl.when(whole)
    def _stage_window():
        a1 = pltpu.async_copy(xl_hbm.at[pl.ds(w0 * NF, (SU + 16) * NF)],
                              xlw, sem_s)
        a2 = pltpu.async_copy(px_hbm.at[pl.ds(aw0, SU + 16)],
                              pwx.at[pl.ds(0, SU + 16)], sem_s)
        a3 = pltpu.async_copy(py_hbm.at[pl.ds(aw0, SU + 16)],
                              pwy.at[pl.ds(0, SU + 16)], sem_s)
        a4 = pltpu.async_copy(pz_hbm.at[pl.ds(aw0, SU + 16)],
                              pwz.at[pl.ds(0, SU + 16)], sem_s)
        a1.wait()
        a2.wait()
        a3.wait()
        a4.wait()

    def graph_body(gl, _):
        s = _sext(sv, gl)
        e = _sext(ev, gl)
        n = e - s

        def vc_body(vc, _):
            v0 = s + vc * CV
            nv = jnp.minimum(CV, n - vc * CV)
            av0 = (v0 // 8) * 8
            voff = v0 - av0

            @pl.when(jnp.logical_not(whole))
            def _stage_vpos():
                d1 = pltpu.async_copy(px_hbm.at[pl.ds(av0, CV + 8)],
                                      pxv.at[pl.ds(0, CV + 8)], sem_s)
                d2 = pltpu.async_copy(py_hbm.at[pl.ds(av0, CV + 8)],
                                      pyv.at[pl.ds(0, CV + 8)], sem_s)
                d3 = pltpu.async_copy(pz_hbm.at[pl.ds(av0, CV + 8)],
                                      pzv.at[pl.ds(0, CV + 8)], sem_s)
                d1.wait()
                d2.wait()
                d3.wait()

            def z_body(ivz, _):
                for c2 in range(8):
                    agg_v[pl.ds(ivz * NF + 16 * c2, 16)] = zero16
                return 0

            lax.fori_loop(0, nv, z_body, 0)

            nuseg = jnp.where(whole, 1, (n + SU - 1) // SU)

            def useg_body(useg, _):
                us0 = s + useg * SU
                nu = jnp.minimum(e, us0 + SU) - us0

                @pl.when(jnp.logical_not(whole))
                def _stage_useg():
                    asu_c = (us0 // 8) * 8
                    e1 = pltpu.async_copy(
                        xl_hbm.at[pl.ds(us0 * NF, (SU + 16) * NF)], xlw,
                        sem_s)
                    e2 = pltpu.async_copy(px_hbm.at[pl.ds(asu_c, SU + 16)],
                                          pwx.at[pl.ds(0, SU + 16)], sem_s)
                    e3 = pltpu.async_copy(py_hbm.at[pl.ds(asu_c, SU + 16)],
                                          pwy.at[pl.ds(0, SU + 16)], sem_s)
                    e4 = pltpu.async_copy(pz_hbm.at[pl.ds(asu_c, SU + 16)],
                                          pwz.at[pl.ds(0, SU + 16)], sem_s)
                    e1.wait()
                    e2.wait()
                    e3.wait()
                    e4.wait()

                asu = jnp.where(whole, aw0, (us0 // 8) * 8)
                pbase = us0 - asu
                xbase0 = jnp.where(whole, us0 - w0, 0)
                nut = (nu + 15) // 16
                vwi = jnp.clip(v0 - aw0, 0, SU + 6)

                def v_body(iv):
                    v = v0 + iv
                    vxw = _sext(pwx, vwi + iv)
                    vyw = _sext(pwy, vwi + iv)
                    vzw = _sext(pwz, vwi + iv)
                    vxc = _sext(pxv, iv + voff)
                    vyc = _sext(pyv, iv + voff)
                    vzc = _sext(pzv, iv + voff)
                    vx = jnp.full((16,), jnp.where(whole, vxw, vxc))
                    vy = jnp.full((16,), jnp.where(whole, vyw, vyc))
                    vz = jnp.full((16,), jnp.where(whole, vzw, vzc))
                    accs = tuple(agg_v[pl.ds(iv * NF + 16 * c2, 16)]
                                 for c2 in range(8))

                    def ut_body(ut, accs):
                        lane0 = 16 * ut
                        px = pwx[pl.ds(pbase + lane0, 16)]
                        py = pwy[pl.ds(pbase + lane0, 16)]
                        pz = pwz[pl.ds(pbase + lane0, 16)]
                        dx = px - vx
                        dy = py - vy
                        dz = pz - vz
                        d2 = dx * dx + dy * dy + dz * dz
                        ul = lane0 + lanes
                        lane_ok = ul < nu
                        d2 = jnp.where(lane_ok, d2, zero16)
                        sel = lane_ok & (d2 <= CUTOFF * CUTOFF) \
                            & ((us0 + ul) != v)
                        x = jnp.maximum(d2, 1e-24)
                        bits = lax.bitcast_convert_type(x, jnp.int32)
                        bits = 0x1FBD1DF5 + (bits >> 1)
                        y = lax.bitcast_convert_type(bits, jnp.float32)
                        y = 0.5 * (y + x / y)
                        y = 0.5 * (y + x / y)
                        kf = jnp.clip(y * ((K - 1) / CUTOFF) + 0.5,
                                      0.0, float(K - 1))
                        ki = kf.astype(jnp.int32) * NF
                        ki = jnp.where(sel, ki, K * NF)
                        acc_l = list(accs)
                        for j in range(16):
                            kj = ki[j]
                            xbase = (xbase0 + lane0 + j) * NF
                            for c2 in range(8):
                                tvec = tab_v[pl.ds(kj + 16 * c2, 16)]
                                xvec = xlw[pl.ds(xbase + 16 * c2, 16)]
                                acc_l[c2] = acc_l[c2] + tvec * xvec
                        return tuple(acc_l)

                    accs = plsc.parallel_loop(0, nut, carry=accs,
                                              unroll=2)(ut_body)
                    for c2 in range(8):
                        agg_v[pl.ds(iv * NF + 16 * c2, 16)] = accs[c2]

                plsc.parallel_loop(0, nv, unroll=2)(v_body)
                return 0

            lax.fori_loop(0, nuseg, useg_body, 0)

            def w_issue(iv, _):
                pltpu.async_copy(agg_v.at[pl.ds(iv * NF, NF)],
                                 agg_hbm.at[pl.ds((v0 + iv) * NF, NF)],
                                 sem_w)
                return 0

            lax.fori_loop(0, nv, w_issue, 0)

            def w_drain(iv, _):
                pltpu.make_async_copy(
                    agg_hbm.at[pl.ds(0, NF)], agg_v.at[pl.ds(0, NF)],
                    sem_w).wait()
                return 0

            lax.fori_loop(0, nv, w_drain, 0)
            return 0

        nvc = (n + CV - 1) // CV
        lax.fori_loop(0, nvc, vc_body, 0)
        return 0

    lax.fori_loop(0, GPT, graph_body, 0)


def _make_msg(starts, ends, px, py, pz, xl_p, tab):
    mesh = plsc.VectorSubcoreMesh(core_axis_name="c", subcore_axis_name="s",
                                  num_cores=NC, num_subcores=NS)
    f = pl.kernel(
        _msg_body,
        out_type=jax.ShapeDtypeStruct((N * NF,), jnp.float32),
        mesh=mesh,
        scratch_types=[
            pltpu.VMEM(((K + 1) * NF,), jnp.float32),
            pltpu.VMEM(((SU + 16) * NF,), jnp.float32),
            pltpu.VMEM((CV * NF,), jnp.float32),
            pltpu.VMEM((SU + 64,), jnp.float32),
            pltpu.VMEM((SU + 64,), jnp.float32),
            pltpu.VMEM((SU + 64,), jnp.float32),
            pltpu.VMEM((64,), jnp.float32),
            pltpu.VMEM((64,), jnp.float32),
            pltpu.VMEM((64,), jnp.float32),
            pltpu.VMEM((GPT + 16,), jnp.int32),
            pltpu.VMEM((GPT + 16,), jnp.int32),
            pltpu.SemaphoreType.DMA,
            pltpu.SemaphoreType.DMA,
        ],
    )
    return f(starts, ends, px, py, pz, xl_p, tab).reshape(N, NF)


def _readout_body(starts_hbm, ends_hbm, y_hbm, out_hbm, y_v, row_v, sv, ev):
    wid = lax.axis_index("s") * NC + lax.axis_index("c")
    base = wid * GPT
    pltpu.sync_copy(starts_hbm.at[pl.ds(base, GPT)], sv.at[pl.ds(0, GPT)])
    pltpu.sync_copy(ends_hbm.at[pl.ds(base, GPT)], ev.at[pl.ds(0, GPT)])
    zero16 = jnp.zeros((16,), jnp.float32)

    def graph_body(gl, _):
        s = _sext(sv, gl)
        e = _sext(ev, gl)
        n = e - s
        nuc = (n + CU - 1) // CU

        def uc_body(uc, accs):
            u0 = s + uc * CU
            nu = jnp.minimum(CU, n - uc * CU)
            pltpu.sync_copy(y_hbm.at[pl.ds(u0 * OUT, CU * OUT)], y_v)

            def r_body(r, accs):
                return tuple(accs[c2] + y_v[pl.ds(r * OUT + 16 * c2, 16)]
                             for c2 in range(8))

            return lax.fori_loop(0, nu, r_body, accs)

        accs = lax.fori_loop(0, nuc, uc_body, tuple(zero16 for _ in range(8)))
        for c2 in range(8):
            row_v[pl.ds(16 * c2, 16)] = accs[c2]
        pltpu.sync_copy(row_v, out_hbm.at[pl.ds((base + gl) * OUT, OUT)])
        return 0

    lax.fori_loop(0, GPT, graph_body, 0)


def _make_readout(starts, ends, y_p):
    mesh = plsc.VectorSubcoreMesh(core_axis_name="c", subcore_axis_name="s",
                                  num_cores=NC, num_subcores=NS)
    f = pl.kernel(
        _readout_body,
        out_type=jax.ShapeDtypeStruct((G * OUT,), jnp.float32),
        mesh=mesh,
        scratch_types=[
            pltpu.VMEM((CU * OUT,), jnp.float32),
            pltpu.VMEM((OUT,), jnp.float32),
            pltpu.VMEM((GPT + 16,), jnp.int32),
            pltpu.VMEM((GPT + 16,), jnp.int32),
        ],
    )
    return f(starts, ends, y_p).reshape(G, OUT)


def kernel(z, pos, batch, emb, Wm1, bm1, Wm2, bm2, Wc1, Wc2, bc2, Wl, bl,
           lin1_W, lin1_b, lin2_W, lin2_b):
    z = z.astype(jnp.int32)
    batch = batch.astype(jnp.int32)
    posf = pos.astype(jnp.float32)
    px = jnp.pad(posf[:, 0], (0, PPAD))
    py = jnp.pad(posf[:, 1], (0, PPAD))
    pz = jnp.pad(posf[:, 2], (0, PPAD))
    batch_t = batch.reshape(16, 512).T
    z2d = z.reshape(N, 1)
    embp = jnp.pad(emb, ((0, 28), (0, 0)))
    wm1p = jnp.pad(Wm1, ((0, 0), (0, NGP - NG), (0, 0)))
    bm1r = bm1.reshape(NI, 1, NF)
    bm2r = bm2.reshape(NI, 1, NF)

    tabs = _make_tables(wm1p, bm1r, Wm2, bm2r)
    starts2, ends2 = _make_bounds(batch_t)
    starts = starts2.reshape(G)
    ends = ends2.reshape(G)

    h, xl = _make_h0_xl(z2d, embp, Wc1[0])
    for i in range(NI):
        xl_p = jnp.pad(xl, ((0, PPAD), (0, 0))).reshape(-1)
        agg = _make_msg(starts, ends, px, py, pz, xl_p, tabs[i].reshape(-1))
        if i < NI - 1:
            h, xl = _make_h_update_xl(agg, h, Wc2[i], bc2[i].reshape(1, H),
                                      Wl[i], bl[i].reshape(1, H), Wc1[i + 1])
        else:
            y = _make_h_update_read(agg, h, Wc2[i], bc2[i].reshape(1, H),
                                    Wl[i], bl[i].reshape(1, H),
                                    lin1_W, lin1_b.reshape(1, H // 2),
                                    lin2_W, lin2_b.reshape(1, OUT))
    y_p = jnp.pad(y, ((0, 64), (0, 0))).reshape(-1)
    return _make_readout(starts, ends, y_p)


# speedup vs baseline: 30.2609x; 1.1034x over previous
import functools

import jax
import jax.numpy as jnp
from jax import lax
from jax.experimental import pallas as pl
from jax.experimental.pallas import tpu as pltpu
from jax.experimental.pallas import tpu_sc as plsc

N = 8192
G = 1024
H = 128
NF = 128
NG = 50
NGP = 64
OUT = 128
NI = 6
CUTOFF = 10.0
K = 512

NC = 2
NS = 16
NW = NC * NS
GPT = G // NW
CV = 32
CU = 32
SU = 384
VOFF = SU + 48
RT = 256
PPAD = SU + 64


def _ssp(x):
    return jax.nn.softplus(x) - jnp.log(2.0)


def _tables_body(wm1_ref, bm1_ref, wm2_ref, bm2_ref, t_ref):
    ki = lax.broadcasted_iota(jnp.int32, (K + 1, 1), 0)
    d = ki.astype(jnp.float32) * (CUTOFF / (K - 1))
    off = lax.broadcasted_iota(jnp.int32, (1, NGP), 1).astype(jnp.float32) \
        * (CUTOFF / (NG - 1))
    step = CUTOFF / (NG - 1)
    coeff = -0.5 / (step * step)
    ea = jnp.exp(coeff * (d - off) ** 2)
    t = _ssp(jnp.dot(ea, wm1_ref[0], preferred_element_type=jnp.float32)
             + bm1_ref[0])
    t = jnp.dot(t, wm2_ref[0], preferred_element_type=jnp.float32) + bm2_ref[0]
    c = 0.5 * (jnp.cos(d * (jnp.pi / CUTOFF)) + 1.0)
    c = jnp.where(ki < K, c, 0.0)
    t_ref[0] = t * c


def _make_tables(wm1p, bm1r, wm2, bm2r):
    return pl.pallas_call(
        _tables_body,
        grid=(NI,),
        in_specs=[
            pl.BlockSpec((1, NGP, NF), lambda i: (i, 0, 0)),
            pl.BlockSpec((1, 1, NF), lambda i: (i, 0, 0)),
            pl.BlockSpec((1, NF, NF), lambda i: (i, 0, 0)),
            pl.BlockSpec((1, 1, NF), lambda i: (i, 0, 0)),
        ],
        out_specs=pl.BlockSpec((1, K + 1, NF), lambda i: (i, 0, 0)),
        out_shape=jax.ShapeDtypeStruct((NI, K + 1, NF), jnp.float32),
    )(wm1p, bm1r, wm2, bm2r)


def _bounds_body(batch_ref, starts_ref, ends_ref):
    gv = lax.broadcasted_iota(jnp.int32, (1, G), 1)
    acc_lt = jnp.zeros((1, G), jnp.int32)
    acc_le = jnp.zeros((1, G), jnp.int32)
    for c in range(16):
        bc = batch_ref[:, pl.ds(c, 1)]
        acc_lt = acc_lt + jnp.sum((bc < gv).astype(jnp.int32), axis=0,
                                  keepdims=True)
        acc_le = acc_le + jnp.sum((bc <= gv).astype(jnp.int32), axis=0,
                                  keepdims=True)
    starts_ref[...] = acc_lt
    ends_ref[...] = acc_le


def _make_bounds(batch_t):
    return pl.pallas_call(
        _bounds_body,
        out_shape=(jax.ShapeDtypeStruct((1, G), jnp.int32),
                   jax.ShapeDtypeStruct((1, G), jnp.int32)),
    )(batch_t)


def _h0x_body(z_ref, emb_ref, wc1_ref, h_ref, xl_ref):
    zt = z_ref[...]
    iot = lax.broadcasted_iota(jnp.int32, (1, 128), 1)
    oneh = (zt == iot).astype(jnp.float32)
    h = jnp.dot(oneh, emb_ref[...], preferred_element_type=jnp.float32)
    h_ref[...] = h
    xl_ref[...] = jnp.dot(h, wc1_ref[...], preferred_element_type=jnp.float32)


def _make_h0_xl(z2d, embp, wc1):
    return pl.pallas_call(
        _h0x_body,
        grid=(N // RT,),
        in_specs=[
            pl.BlockSpec((RT, 1), lambda i: (i, 0)),
            pl.BlockSpec((128, 128), lambda i: (0, 0)),
            pl.BlockSpec((H, NF), lambda i: (0, 0)),
        ],
        out_specs=[
            pl.BlockSpec((RT, 128), lambda i: (i, 0)),
            pl.BlockSpec((RT, NF), lambda i: (i, 0)),
        ],
        out_shape=[jax.ShapeDtypeStruct((N, 128), jnp.float32),
                   jax.ShapeDtypeStruct((N, NF), jnp.float32)],
    )(z2d, embp, wc1)


def _khx_body(agg_ref, h_ref, wc2_ref, bc2_ref, wl_ref, bl_ref, wc1_ref,
              hout_ref, xl_ref):
    x2 = jnp.dot(agg_ref[...], wc2_ref[...],
                 preferred_element_type=jnp.float32) + bc2_ref[...]
    x2 = _ssp(x2)
    x2 = jnp.dot(x2, wl_ref[...], preferred_element_type=jnp.float32) \
        + bl_ref[...]
    h = h_ref[...] + x2
    hout_ref[...] = h
    xl_ref[...] = jnp.dot(h, wc1_ref[...], preferred_element_type=jnp.float32)


def _make_h_update_xl(agg, h, wc2, bc2r, wl, blr, wc1n):
    return pl.pallas_call(
        _khx_body,
        grid=(N // RT,),
        in_specs=[
            pl.BlockSpec((RT, NF), lambda i: (i, 0)),
            pl.BlockSpec((RT, H), lambda i: (i, 0)),
            pl.BlockSpec((NF, H), lambda i: (0, 0)),
            pl.BlockSpec((1, H), lambda i: (0, 0)),
            pl.BlockSpec((H, H), lambda i: (0, 0)),
            pl.BlockSpec((1, H), lambda i: (0, 0)),
            pl.BlockSpec((H, NF), lambda i: (0, 0)),
        ],
        out_specs=[
            pl.BlockSpec((RT, H), lambda i: (i, 0)),
            pl.BlockSpec((RT, NF), lambda i: (i, 0)),
        ],
        out_shape=[jax.ShapeDtypeStruct((N, H), jnp.float32),
                   jax.ShapeDtypeStruct((N, NF), jnp.float32)],
    )(agg, h, wc2, bc2r, wl, blr, wc1n)


def _khread_body(agg_ref, h_ref, wc2_ref, bc2_ref, wl_ref, bl_ref,
                 w1_ref, b1_ref, w2_ref, b2_ref, y_ref):
    x2 = jnp.dot(agg_ref[...], wc2_ref[...],
                 preferred_element_type=jnp.float32) + bc2_ref[...]
    x2 = _ssp(x2)
    x2 = jnp.dot(x2, wl_ref[...], preferred_element_type=jnp.float32) \
        + bl_ref[...]
    h = h_ref[...] + x2
    t = _ssp(jnp.dot(h, w1_ref[...],
                     preferred_element_type=jnp.float32) + b1_ref[...])
    y_ref[...] = jnp.dot(t, w2_ref[...],
                         preferred_element_type=jnp.float32) + b2_ref[...]


def _make_h_update_read(agg, h, wc2, bc2r, wl, blr, w1, b1r, w2, b2r):
    return pl.pallas_call(
        _khread_body,
        grid=(N // RT,),
        in_specs=[
            pl.BlockSpec((RT, NF), lambda i: (i, 0)),
            pl.BlockSpec((RT, H), lambda i: (i, 0)),
            pl.BlockSpec((NF, H), lambda i: (0, 0)),
            pl.BlockSpec((1, H), lambda i: (0, 0)),
            pl.BlockSpec((H, H), lambda i: (0, 0)),
            pl.BlockSpec((1, H), lambda i: (0, 0)),
            pl.BlockSpec((H, H // 2), lambda i: (0, 0)),
            pl.BlockSpec((1, H // 2), lambda i: (0, 0)),
            pl.BlockSpec((H // 2, OUT), lambda i: (0, 0)),
            pl.BlockSpec((1, OUT), lambda i: (0, 0)),
        ],
        out_specs=pl.BlockSpec((RT, OUT), lambda i: (i, 0)),
        out_shape=jax.ShapeDtypeStruct((N, OUT), jnp.float32),
    )(agg, h, wc2, bc2r, wl, blr, w1, b1r, w2, b2r)


def _sext(buf, i):
    return buf[pl.ds(i, 16)][0]


def _msg_body(starts_hbm, ends_hbm, px_hbm, py_hbm, pz_hbm, xl_hbm, tab_hbm,
              agg_hbm, tab_v, xlw, agg_v, pwx, pwy, pwz,
              sv, ev, sem_s, sem_w):
    wid = lax.axis_index("s") * NC + lax.axis_index("c")
    base = wid * GPT
    pltpu.sync_copy(tab_hbm, tab_v)
    pltpu.sync_copy(starts_hbm.at[pl.ds(base, GPT)], sv.at[pl.ds(0, GPT)])
    pltpu.sync_copy(ends_hbm.at[pl.ds(base, GPT)], ev.at[pl.ds(0, GPT)])
    lanes = lax.iota(jnp.int32, 16)
    zero16 = jnp.zeros((16,), jnp.float32)

    w0 = _sext(sv, 0)
    w1 = _sext(ev, GPT - 1)
    wlen = w1 - w0
    whole = wlen <= SU
    aw0 = (w0 // 8) * 8

    @pl.when(whole)
    def _stage_window():
        a1 = pltpu.async_copy(xl_hbm.at[pl.ds(w0 * NF, (SU + 16) * NF)],
                              xlw, sem_s)
        a2 = pltpu.async_copy(px_hbm.at[pl.ds(aw0, SU + 16)],
                              pwx.at[pl.ds(0, SU + 16)], sem_s)
        a3 = pltpu.async_copy(py_hbm.at[pl.ds(aw0, SU + 16)],
                              pwy.at[pl.ds(0, SU + 16)], sem_s)
        a4 = pltpu.async_copy(pz_hbm.at[pl.ds(aw0, SU + 16)],
                              pwz.at[pl.ds(0, SU + 16)], sem_s)
        a1.wait()
        a2.wait()
        a3.wait()
        a4.wait()

    def graph_body(gl, _):
        s = _sext(sv, gl)
        e = _sext(ev, gl)
        n = e - s

        def vc_body(vc, _):
            v0 = s + vc * CV
            nv = jnp.minimum(CV, n - vc * CV)
            av0 = (v0 // 8) * 8
            voff = v0 - av0

            @pl.when(jnp.logical_not(whole))
            def _stage_vpos():
                d1 = pltpu.async_copy(px_hbm.at[pl.ds(av0, CV + 8)],
                                      pwx.at[pl.ds(VOFF, CV + 8)], sem_s)
                d2 = pltpu.async_copy(py_hbm.at[pl.ds(av0, CV + 8)],
                                      pwy.at[pl.ds(VOFF, CV + 8)], sem_s)
                d3 = pltpu.async_copy(pz_hbm.at[pl.ds(av0, CV + 8)],
                                      pwz.at[pl.ds(VOFF, CV + 8)], sem_s)
                d1.wait()
                d2.wait()
                d3.wait()

            def z_body(ivz, _):
                for c2 in range(8):
                    agg_v[pl.ds(ivz * NF + 16 * c2, 16)] = zero16
                return 0

            lax.fori_loop(0, nv, z_body, 0)

            nuseg = jnp.where(whole, 1, (n + SU - 1) // SU)

            def useg_body(useg, _):
                us0 = s + useg * SU
                nu = jnp.minimum(e, us0 + SU) - us0

                @pl.when(jnp.logical_not(whole))
                def _stage_useg():
                    asu_c = (us0 // 8) * 8
                    e1 = pltpu.async_copy(
                        xl_hbm.at[pl.ds(us0 * NF, (SU + 16) * NF)], xlw,
                        sem_s)
                    e2 = pltpu.async_copy(px_hbm.at[pl.ds(asu_c, SU + 16)],
                                          pwx.at[pl.ds(0, SU + 16)], sem_s)
                    e3 = pltpu.async_copy(py_hbm.at[pl.ds(asu_c, SU + 16)],
                                          pwy.at[pl.ds(0, SU + 16)], sem_s)
                    e4 = pltpu.async_copy(pz_hbm.at[pl.ds(asu_c, SU + 16)],
                                          pwz.at[pl.ds(0, SU + 16)], sem_s)
                    e1.wait()
                    e2.wait()
                    e3.wait()
                    e4.wait()

                asu = jnp.where(whole, aw0, (us0 // 8) * 8)
                pbase = us0 - asu
                xbase0 = jnp.where(whole, us0 - w0, 0)
                nut = (nu + 15) // 16
                vwi = jnp.where(whole, jnp.clip(v0 - aw0, 0, SU + 6),
                                VOFF + voff)

                def v_body(iv):
                    v = v0 + iv
                    vx = jnp.full((16,), _sext(pwx, vwi + iv))
                    vy = jnp.full((16,), _sext(pwy, vwi + iv))
                    vz = jnp.full((16,), _sext(pwz, vwi + iv))
                    accs = tuple(agg_v[pl.ds(iv * NF + 16 * c2, 16)]
                                 for c2 in range(8))

                    def ut_body(ut, accs):
                        lane0 = 16 * ut
                        px = pwx[pl.ds(pbase + lane0, 16)]
                        py = pwy[pl.ds(pbase + lane0, 16)]
                        pz = pwz[pl.ds(pbase + lane0, 16)]
                        dx = px - vx
                        dy = py - vy
                        dz = pz - vz
                        d2 = dx * dx + dy * dy + dz * dz
                        ul = lane0 + lanes
                        lane_ok = ul < nu
                        d2 = jnp.where(lane_ok, d2, zero16)
                        sel = lane_ok & (d2 <= CUTOFF * CUTOFF) \
                            & ((us0 + ul) != v)
                        x = jnp.maximum(d2, 1e-24)
                        bits = lax.bitcast_convert_type(x, jnp.int32)
                        bits = 0x1FBD1DF5 + (bits >> 1)
                        y = lax.bitcast_convert_type(bits, jnp.float32)
                        y = 0.5 * (y + x / y)
                        y = 0.5 * (y + x / y)
                        kf = jnp.clip(y * ((K - 1) / CUTOFF) + 0.5,
                                      0.0, float(K - 1))
                        ki = kf.astype(jnp.int32) * NF
                        ki = jnp.where(sel, ki, K * NF)
                        acc_l = list(accs)
                        for j in range(16):
                            kj = ki[j]
                            xbase = (xbase0 + lane0 + j) * NF
                            for c2 in range(8):
                                tvec = tab_v[pl.ds(kj + 16 * c2, 16)]
                                xvec = xlw[pl.ds(xbase + 16 * c2, 16)]
                                acc_l[c2] = acc_l[c2] + tvec * xvec
                        return tuple(acc_l)

                    accs = plsc.parallel_loop(0, nut, carry=accs,
                                              unroll=2)(ut_body)
                    for c2 in range(8):
                        agg_v[pl.ds(iv * NF + 16 * c2, 16)] = accs[c2]

                plsc.parallel_loop(0, nv, unroll=2)(v_body)
                return 0

            lax.fori_loop(0, nuseg, useg_body, 0)

            def w_issue(iv, _):
                pltpu.async_copy(agg_v.at[pl.ds(iv * NF, NF)],
                                 agg_hbm.at[pl.ds((v0 + iv) * NF, NF)],
                                 sem_w)
                return 0

            lax.fori_loop(0, nv, w_issue, 0)

            def w_drain(iv, _):
                pltpu.make_async_copy(
                    agg_hbm.at[pl.ds(0, NF)], agg_v.at[pl.ds(0, NF)],
                    sem_w).wait()
                return 0

            lax.fori_loop(0, nv, w_drain, 0)
            return 0

        nvc = (n + CV - 1) // CV
        lax.fori_loop(0, nvc, vc_body, 0)
        return 0

    lax.fori_loop(0, GPT, graph_body, 0)


def _make_msg(starts, ends, px, py, pz, xl_p, tab):
    mesh = plsc.VectorSubcoreMesh(core_axis_name="c", subcore_axis_name="s",
                                  num_cores=NC, num_subcores=NS)
    f = pl.kernel(
        _msg_body,
        out_type=jax.ShapeDtypeStruct((N * NF,), jnp.float32),
        mesh=mesh,
        scratch_types=[
            pltpu.VMEM(((K + 1) * NF,), jnp.float32),
            pltpu.VMEM(((SU + 16) * NF,), jnp.float32),
            pltpu.VMEM((CV * NF,), jnp.float32),
            pltpu.VMEM((SU + 112,), jnp.float32),
            pltpu.VMEM((SU + 112,), jnp.float32),
            pltpu.VMEM((SU + 112,), jnp.float32),
            pltpu.VMEM((GPT + 16,), jnp.int32),
            pltpu.VMEM((GPT + 16,), jnp.int32),
            pltpu.SemaphoreType.DMA,
            pltpu.SemaphoreType.DMA,
        ],
    )
    return f(starts, ends, px, py, pz, xl_p, tab).reshape(N, NF)


def _readout_body(starts_hbm, ends_hbm, y_hbm, out_hbm, y_v, row_v, sv, ev):
    wid = lax.axis_index("s") * NC + lax.axis_index("c")
    base = wid * GPT
    pltpu.sync_copy(starts_hbm.at[pl.ds(base, GPT)], sv.at[pl.ds(0, GPT)])
    pltpu.sync_copy(ends_hbm.at[pl.ds(base, GPT)], ev.at[pl.ds(0, GPT)])
    zero16 = jnp.zeros((16,), jnp.float32)

    def graph_body(gl, _):
        s = _sext(sv, gl)
        e = _sext(ev, gl)
        n = e - s
        nuc = (n + CU - 1) // CU

        def uc_body(uc, accs):
            u0 = s + uc * CU
            nu = jnp.minimum(CU, n - uc * CU)
            pltpu.sync_copy(y_hbm.at[pl.ds(u0 * OUT, CU * OUT)], y_v)

            def r_body(r, accs):
                return tuple(accs[c2] + y_v[pl.ds(r * OUT + 16 * c2, 16)]
                             for c2 in range(8))

            return lax.fori_loop(0, nu, r_body, accs)

        accs = lax.fori_loop(0, nuc, uc_body, tuple(zero16 for _ in range(8)))
        for c2 in range(8):
            row_v[pl.ds(16 * c2, 16)] = accs[c2]
        pltpu.sync_copy(row_v, out_hbm.at[pl.ds((base + gl) * OUT, OUT)])
        return 0

    lax.fori_loop(0, GPT, graph_body, 0)


def _make_readout(starts, ends, y_p):
    mesh = plsc.VectorSubcoreMesh(core_axis_name="c", subcore_axis_name="s",
                                  num_cores=NC, num_subcores=NS)
    f = pl.kernel(
        _readout_body,
        out_type=jax.ShapeDtypeStruct((G * OUT,), jnp.float32),
        mesh=mesh,
        scratch_types=[
            pltpu.VMEM((CU * OUT,), jnp.float32),
            pltpu.VMEM((OUT,), jnp.float32),
            pltpu.VMEM((GPT + 16,), jnp.int32),
            pltpu.VMEM((GPT + 16,), jnp.int32),
        ],
    )
    return f(starts, ends, y_p).reshape(G, OUT)


def kernel(z, pos, batch, emb, Wm1, bm1, Wm2, bm2, Wc1, Wc2, bc2, Wl, bl,
           lin1_W, lin1_b, lin2_W, lin2_b):
    z = z.astype(jnp.int32)
    batch = batch.astype(jnp.int32)
    posf = pos.astype(jnp.float32)
    px = jnp.pad(posf[:, 0], (0, PPAD))
    py = jnp.pad(posf[:, 1], (0, PPAD))
    pz = jnp.pad(posf[:, 2], (0, PPAD))
    batch_t = batch.reshape(16, 512).T
    z2d = z.reshape(N, 1)
    embp = jnp.pad(emb, ((0, 28), (0, 0)))
    wm1p = jnp.pad(Wm1, ((0, 0), (0, NGP - NG), (0, 0)))
    bm1r = bm1.reshape(NI, 1, NF)
    bm2r = bm2.reshape(NI, 1, NF)

    tabs = _make_tables(wm1p, bm1r, Wm2, bm2r)
    starts2, ends2 = _make_bounds(batch_t)
    starts = starts2.reshape(G)
    ends = ends2.reshape(G)

    h, xl = _make_h0_xl(z2d, embp, Wc1[0])
    for i in range(NI):
        xl_p = jnp.pad(xl, ((0, PPAD), (0, 0))).reshape(-1)
        agg = _make_msg(starts, ends, px, py, pz, xl_p, tabs[i].reshape(-1))
        if i < NI - 1:
            h, xl = _make_h_update_xl(agg, h, Wc2[i], bc2[i].reshape(1, H),
                                      Wl[i], bl[i].reshape(1, H), Wc1[i + 1])
        else:
            y = _make_h_update_read(agg, h, Wc2[i], bc2[i].reshape(1, H),
                                    Wl[i], bl[i].reshape(1, H),
                                    lin1_W, lin1_b.reshape(1, H // 2),
                                    lin2_W, lin2_b.reshape(1, OUT))
    y_p = jnp.pad(y, ((0, 64), (0, 0))).reshape(-1)
    return _make_readout(starts, ends, y_p)


# speedup vs baseline: 31.9472x; 1.0557x over previous
import functools

import jax
import jax.numpy as jnp
from jax import lax
from jax.experimental import pallas as pl
from jax.experimental.pallas import tpu as pltpu
from jax.experimental.pallas import tpu_sc as plsc

N = 8192
G = 1024
H = 128
NF = 128
NG = 50
NGP = 64
OUT = 128
NI = 6
CUTOFF = 10.0
K = 512

NC = 2
NS = 16
NW = NC * NS
GPT = G // NW
CV = 32
CU = 32
SU = 384
VOFF = SU + 48
RT = 256
PPAD = SU + 64


def _ssp(x):
    return jax.nn.softplus(x) - jnp.log(2.0)


def _tables_body(wm1_ref, bm1_ref, wm2_ref, bm2_ref, t_ref):
    ki = lax.broadcasted_iota(jnp.int32, (K + 1, 1), 0)
    d = ki.astype(jnp.float32) * (CUTOFF / (K - 1))
    off = lax.broadcasted_iota(jnp.int32, (1, NGP), 1).astype(jnp.float32) \
        * (CUTOFF / (NG - 1))
    step = CUTOFF / (NG - 1)
    coeff = -0.5 / (step * step)
    ea = jnp.exp(coeff * (d - off) ** 2)
    t = _ssp(jnp.dot(ea, wm1_ref[0], preferred_element_type=jnp.float32)
             + bm1_ref[0])
    t = jnp.dot(t, wm2_ref[0], preferred_element_type=jnp.float32) + bm2_ref[0]
    c = 0.5 * (jnp.cos(d * (jnp.pi / CUTOFF)) + 1.0)
    c = jnp.where(ki < K, c, 0.0)
    t_ref[0] = t * c


def _make_tables(wm1p, bm1r, wm2, bm2r):
    return pl.pallas_call(
        _tables_body,
        grid=(NI,),
        in_specs=[
            pl.BlockSpec((1, NGP, NF), lambda i: (i, 0, 0)),
            pl.BlockSpec((1, 1, NF), lambda i: (i, 0, 0)),
            pl.BlockSpec((1, NF, NF), lambda i: (i, 0, 0)),
            pl.BlockSpec((1, 1, NF), lambda i: (i, 0, 0)),
        ],
        out_specs=pl.BlockSpec((1, K + 1, NF), lambda i: (i, 0, 0)),
        out_shape=jax.ShapeDtypeStruct((NI, K + 1, NF), jnp.float32),
    )(wm1p, bm1r, wm2, bm2r)


def _bounds_body(batch_ref, starts_ref, ends_ref):
    gv = lax.broadcasted_iota(jnp.int32, (1, G), 1)
    acc_lt = jnp.zeros((1, G), jnp.int32)
    acc_le = jnp.zeros((1, G), jnp.int32)
    for c in range(16):
        bc = batch_ref[:, pl.ds(c, 1)]
        acc_lt = acc_lt + jnp.sum((bc < gv).astype(jnp.int32), axis=0,
                                  keepdims=True)
        acc_le = acc_le + jnp.sum((bc <= gv).astype(jnp.int32), axis=0,
                                  keepdims=True)
    starts_ref[...] = acc_lt
    ends_ref[...] = acc_le


def _make_bounds(batch_t):
    return pl.pallas_call(
        _bounds_body,
        out_shape=(jax.ShapeDtypeStruct((1, G), jnp.int32),
                   jax.ShapeDtypeStruct((1, G), jnp.int32)),
    )(batch_t)


def _h0x_body(z_ref, emb_ref, wc1_ref, h_ref, xl_ref):
    zt = z_ref[...]
    iot = lax.broadcasted_iota(jnp.int32, (1, 128), 1)
    oneh = (zt == iot).astype(jnp.float32)
    h = jnp.dot(oneh, emb_ref[...], preferred_element_type=jnp.float32)
    h_ref[...] = h
    xl_ref[...] = jnp.dot(h, wc1_ref[...], preferred_element_type=jnp.float32)


def _make_h0_xl(z2d, embp, wc1):
    return pl.pallas_call(
        _h0x_body,
        grid=(N // RT,),
        in_specs=[
            pl.BlockSpec((RT, 1), lambda i: (i, 0)),
            pl.BlockSpec((128, 128), lambda i: (0, 0)),
            pl.BlockSpec((H, NF), lambda i: (0, 0)),
        ],
        out_specs=[
            pl.BlockSpec((RT, 128), lambda i: (i, 0)),
            pl.BlockSpec((RT, NF), lambda i: (i, 0)),
        ],
        out_shape=[jax.ShapeDtypeStruct((N, 128), jnp.float32),
                   jax.ShapeDtypeStruct((N, NF), jnp.float32)],
    )(z2d, embp, wc1)


def _khx_body(agg_ref, h_ref, wc2_ref, bc2_ref, wl_ref, bl_ref, wc1_ref,
              hout_ref, xl_ref):
    x2 = jnp.dot(agg_ref[...], wc2_ref[...],
                 preferred_element_type=jnp.float32) + bc2_ref[...]
    x2 = _ssp(x2)
    x2 = jnp.dot(x2, wl_ref[...], preferred_element_type=jnp.float32) \
        + bl_ref[...]
    h = h_ref[...] + x2
    hout_ref[...] = h
    xl_ref[...] = jnp.dot(h, wc1_ref[...], preferred_element_type=jnp.float32)


def _make_h_update_xl(agg, h, wc2, bc2r, wl, blr, wc1n):
    return pl.pallas_call(
        _khx_body,
        grid=(N // RT,),
        in_specs=[
            pl.BlockSpec((RT, NF), lambda i: (i, 0)),
            pl.BlockSpec((RT, H), lambda i: (i, 0)),
            pl.BlockSpec((NF, H), lambda i: (0, 0)),
            pl.BlockSpec((1, H), lambda i: (0, 0)),
            pl.BlockSpec((H, H), lambda i: (0, 0)),
            pl.BlockSpec((1, H), lambda i: (0, 0)),
            pl.BlockSpec((H, NF), lambda i: (0, 0)),
        ],
        out_specs=[
            pl.BlockSpec((RT, H), lambda i: (i, 0)),
            pl.BlockSpec((RT, NF), lambda i: (i, 0)),
        ],
        out_shape=[jax.ShapeDtypeStruct((N, H), jnp.float32),
                   jax.ShapeDtypeStruct((N, NF), jnp.float32)],
    )(agg, h, wc2, bc2r, wl, blr, wc1n)


def _khread_body(agg_ref, h_ref, wc2_ref, bc2_ref, wl_ref, bl_ref,
                 w1_ref, b1_ref, w2_ref, b2_ref, y_ref):
    x2 = jnp.dot(agg_ref[...], wc2_ref[...],
                 preferred_element_type=jnp.float32) + bc2_ref[...]
    x2 = _ssp(x2)
    x2 = jnp.dot(x2, wl_ref[...], preferred_element_type=jnp.float32) \
        + bl_ref[...]
    h = h_ref[...] + x2
    t = _ssp(jnp.dot(h, w1_ref[...],
                     preferred_element_type=jnp.float32) + b1_ref[...])
    y_ref[...] = jnp.dot(t, w2_ref[...],
                         preferred_element_type=jnp.float32) + b2_ref[...]


def _make_h_update_read(agg, h, wc2, bc2r, wl, blr, w1, b1r, w2, b2r):
    return pl.pallas_call(
        _khread_body,
        grid=(N // RT,),
        in_specs=[
            pl.BlockSpec((RT, NF), lambda i: (i, 0)),
            pl.BlockSpec((RT, H), lambda i: (i, 0)),
            pl.BlockSpec((NF, H), lambda i: (0, 0)),
            pl.BlockSpec((1, H), lambda i: (0, 0)),
            pl.BlockSpec((H, H), lambda i: (0, 0)),
            pl.BlockSpec((1, H), lambda i: (0, 0)),
            pl.BlockSpec((H, H // 2), lambda i: (0, 0)),
            pl.BlockSpec((1, H // 2), lambda i: (0, 0)),
            pl.BlockSpec((H // 2, OUT), lambda i: (0, 0)),
            pl.BlockSpec((1, OUT), lambda i: (0, 0)),
        ],
        out_specs=pl.BlockSpec((RT, OUT), lambda i: (i, 0)),
        out_shape=jax.ShapeDtypeStruct((N, OUT), jnp.float32),
    )(agg, h, wc2, bc2r, wl, blr, w1, b1r, w2, b2r)


def _sext(buf, i):
    return buf[pl.ds(i, 16)][0]


def _msg_body(starts_hbm, ends_hbm, px_hbm, py_hbm, pz_hbm, xl_hbm, tab_hbm,
              agg_hbm, tab_v, xlw, agg_v, pwx, pwy, pwz,
              sv, ev, sem_s, sem_w):
    wid = lax.axis_index("s") * NC + lax.axis_index("c")
    base = wid * GPT
    pltpu.sync_copy(tab_hbm, tab_v)
    pltpu.sync_copy(starts_hbm.at[pl.ds(base, GPT)], sv.at[pl.ds(0, GPT)])
    pltpu.sync_copy(ends_hbm.at[pl.ds(base, GPT)], ev.at[pl.ds(0, GPT)])
    lanes = lax.iota(jnp.int32, 16)
    zero16 = jnp.zeros((16,), jnp.float32)

    w0 = _sext(sv, 0)
    w1 = _sext(ev, GPT - 1)
    wlen = w1 - w0
    whole = wlen <= SU
    aw0 = (w0 // 8) * 8

    @pl.when(whole)
    def _stage_window():
        a1 = pltpu.async_copy(xl_hbm.at[pl.ds(w0 * NF, (SU + 16) * NF)],
                              xlw, sem_s)
        a2 = pltpu.async_copy(px_hbm.at[pl.ds(aw0, SU + 16)],
                              pwx.at[pl.ds(0, SU + 16)], sem_s)
        a3 = pltpu.async_copy(py_hbm.at[pl.ds(aw0, SU + 16)],
                              pwy.at[pl.ds(0, SU + 16)], sem_s)
        a4 = pltpu.async_copy(pz_hbm.at[pl.ds(aw0, SU + 16)],
                              pwz.at[pl.ds(0, SU + 16)], sem_s)
        a1.wait()
        a2.wait()
        a3.wait()
        a4.wait()

    def graph_body(gl, _):
        s = _sext(sv, gl)
        e = _sext(ev, gl)
        n = e - s

        def vc_body(vc, _):
            v0 = s + vc * CV
            nv = jnp.minimum(CV, n - vc * CV)
            av0 = (v0 // 8) * 8
            voff = v0 - av0

            @pl.when(jnp.logical_not(whole))
            def _stage_vpos():
                d1 = pltpu.async_copy(px_hbm.at[pl.ds(av0, CV + 8)],
                                      pwx.at[pl.ds(VOFF, CV + 8)], sem_s)
                d2 = pltpu.async_copy(py_hbm.at[pl.ds(av0, CV + 8)],
                                      pwy.at[pl.ds(VOFF, CV + 8)], sem_s)
                d3 = pltpu.async_copy(pz_hbm.at[pl.ds(av0, CV + 8)],
                                      pwz.at[pl.ds(VOFF, CV + 8)], sem_s)
                d1.wait()
                d2.wait()
                d3.wait()

            def z_body(ivz, _):
                for c2 in range(8):
                    agg_v[pl.ds(ivz * NF + 16 * c2, 16)] = zero16
                return 0

            lax.fori_loop(0, nv, z_body, 0)

            nuseg = jnp.where(whole, 1, (n + SU - 1) // SU)

            def useg_body(useg, _):
                us0 = s + useg * SU
                nu = jnp.minimum(e, us0 + SU) - us0

                @pl.when(jnp.logical_not(whole))
                def _stage_useg():
                    asu_c = (us0 // 8) * 8
                    e1 = pltpu.async_copy(
                        xl_hbm.at[pl.ds(us0 * NF, (SU + 16) * NF)], xlw,
                        sem_s)
                    e2 = pltpu.async_copy(px_hbm.at[pl.ds(asu_c, SU + 16)],
                                          pwx.at[pl.ds(0, SU + 16)], sem_s)
                    e3 = pltpu.async_copy(py_hbm.at[pl.ds(asu_c, SU + 16)],
                                          pwy.at[pl.ds(0, SU + 16)], sem_s)
                    e4 = pltpu.async_copy(pz_hbm.at[pl.ds(asu_c, SU + 16)],
                                          pwz.at[pl.ds(0, SU + 16)], sem_s)
                    e1.wait()
                    e2.wait()
                    e3.wait()
                    e4.wait()

                asu = jnp.where(whole, aw0, (us0 // 8) * 8)
                pbase = us0 - asu
                xbase0 = jnp.where(whole, us0 - w0, 0)
                nut = (nu + 15) // 16
                vwi = jnp.where(whole, jnp.clip(v0 - aw0, 0, SU + 6),
                                VOFF + voff)

                def vq_body(ivq):
                    iv0 = 2 * ivq
                    vs = [v0 + iv0 + q for q in range(2)]
                    vxs = [jnp.full((16,), _sext(pwx, vwi + iv0 + q))
                           for q in range(2)]
                    vys = [jnp.full((16,), _sext(pwy, vwi + iv0 + q))
                           for q in range(2)]
                    vzs = [jnp.full((16,), _sext(pwz, vwi + iv0 + q))
                           for q in range(2)]
                    accs = tuple(
                        agg_v[pl.ds((iv0 + q) * NF + 16 * c2, 16)]
                        for q in range(2) for c2 in range(8))

                    def ut_body(ut, accs):
                        lane0 = 16 * ut
                        px = pwx[pl.ds(pbase + lane0, 16)]
                        py = pwy[pl.ds(pbase + lane0, 16)]
                        pz = pwz[pl.ds(pbase + lane0, 16)]
                        ul = lane0 + lanes
                        lane_ok = ul < nu
                        uglob = us0 + ul
                        kis = []
                        for q in range(2):
                            dx = px - vxs[q]
                            dy = py - vys[q]
                            dz = pz - vzs[q]
                            d2 = dx * dx + dy * dy + dz * dz
                            d2 = jnp.where(lane_ok, d2, zero16)
                            sel = lane_ok & (d2 <= CUTOFF * CUTOFF) \
                                & (uglob != vs[q])
                            x = jnp.maximum(d2, 1e-24)
                            bits = lax.bitcast_convert_type(x, jnp.int32)
                            bits = 0x1FBD1DF5 + (bits >> 1)
                            y = lax.bitcast_convert_type(bits, jnp.float32)
                            y = 0.5 * (y + x / y)
                            y = 0.5 * (y + x / y)
                            kf = jnp.clip(y * ((K - 1) / CUTOFF) + 0.5,
                                          0.0, float(K - 1))
                            ki = kf.astype(jnp.int32) * NF
                            kis.append(jnp.where(sel, ki, K * NF))
                        acc_l = list(accs)
                        for j in range(16):
                            kjs = [kis[q][j] for q in range(2)]
                            xbase = (xbase0 + lane0 + j) * NF
                            for c2 in range(8):
                                xvec = xlw[pl.ds(xbase + 16 * c2, 16)]
                                for q in range(2):
                                    tvec = tab_v[pl.ds(kjs[q] + 16 * c2, 16)]
                                    acc_l[8 * q + c2] = \
                                        acc_l[8 * q + c2] + tvec * xvec
                        return tuple(acc_l)

                    accs = plsc.parallel_loop(0, nut, carry=accs,
                                              unroll=1)(ut_body)
                    for q in range(2):
                        for c2 in range(8):
                            agg_v[pl.ds((iv0 + q) * NF + 16 * c2, 16)] = \
                                accs[8 * q + c2]

                plsc.parallel_loop(0, (nv + 1) // 2, unroll=1)(vq_body)
                return 0

            lax.fori_loop(0, nuseg, useg_body, 0)

            def w_issue(iv, _):
                pltpu.async_copy(agg_v.at[pl.ds(iv * NF, NF)],
                                 agg_hbm.at[pl.ds((v0 + iv) * NF, NF)],
                                 sem_w)
                return 0

            lax.fori_loop(0, nv, w_issue, 0)

            def w_drain(iv, _):
                pltpu.make_async_copy(
                    agg_hbm.at[pl.ds(0, NF)], agg_v.at[pl.ds(0, NF)],
                    sem_w).wait()
                return 0

            lax.fori_loop(0, nv, w_drain, 0)
            return 0

        nvc = (n + CV - 1) // CV
        lax.fori_loop(0, nvc, vc_body, 0)
        return 0

    lax.fori_loop(0, GPT, graph_body, 0)


def _make_msg(starts, ends, px, py, pz, xl_p, tab):
    mesh = plsc.VectorSubcoreMesh(core_axis_name="c", subcore_axis_name="s",
                                  num_cores=NC, num_subcores=NS)
    f = pl.kernel(
        _msg_body,
        out_type=jax.ShapeDtypeStruct((N * NF,), jnp.float32),
        mesh=mesh,
        scratch_types=[
            pltpu.VMEM(((K + 1) * NF,), jnp.float32),
            pltpu.VMEM(((SU + 16) * NF,), jnp.float32),
            pltpu.VMEM((CV * NF,), jnp.float32),
            pltpu.VMEM((SU + 112,), jnp.float32),
            pltpu.VMEM((SU + 112,), jnp.float32),
            pltpu.VMEM((SU + 112,), jnp.float32),
            pltpu.VMEM((GPT + 16,), jnp.int32),
            pltpu.VMEM((GPT + 16,), jnp.int32),
            pltpu.SemaphoreType.DMA,
            pltpu.SemaphoreType.DMA,
        ],
    )
    return f(starts, ends, px, py, pz, xl_p, tab).reshape(N, NF)


def _readout_body(starts_hbm, ends_hbm, y_hbm, out_hbm, y_v, row_v, sv, ev):
    wid = lax.axis_index("s") * NC + lax.axis_index("c")
    base = wid * GPT
    pltpu.sync_copy(starts_hbm.at[pl.ds(base, GPT)], sv.at[pl.ds(0, GPT)])
    pltpu.sync_copy(ends_hbm.at[pl.ds(base, GPT)], ev.at[pl.ds(0, GPT)])
    zero16 = jnp.zeros((16,), jnp.float32)

    def graph_body(gl, _):
        s = _sext(sv, gl)
        e = _sext(ev, gl)
        n = e - s
        nuc = (n + CU - 1) // CU

        def uc_body(uc, accs):
            u0 = s + uc * CU
            nu = jnp.minimum(CU, n - uc * CU)
            pltpu.sync_copy(y_hbm.at[pl.ds(u0 * OUT, CU * OUT)], y_v)

            def r_body(r, accs):
                return tuple(accs[c2] + y_v[pl.ds(r * OUT + 16 * c2, 16)]
                             for c2 in range(8))

            return lax.fori_loop(0, nu, r_body, accs)

        accs = lax.fori_loop(0, nuc, uc_body, tuple(zero16 for _ in range(8)))
        for c2 in range(8):
            row_v[pl.ds(16 * c2, 16)] = accs[c2]
        pltpu.sync_copy(row_v, out_hbm.at[pl.ds((base + gl) * OUT, OUT)])
        return 0

    lax.fori_loop(0, GPT, graph_body, 0)


def _make_readout(starts, ends, y_p):
    mesh = plsc.VectorSubcoreMesh(core_axis_name="c", subcore_axis_name="s",
                                  num_cores=NC, num_subcores=NS)
    f = pl.kernel(
        _readout_body,
        out_type=jax.ShapeDtypeStruct((G * OUT,), jnp.float32),
        mesh=mesh,
        scratch_types=[
            pltpu.VMEM((CU * OUT,), jnp.float32),
            pltpu.VMEM((OUT,), jnp.float32),
            pltpu.VMEM((GPT + 16,), jnp.int32),
            pltpu.VMEM((GPT + 16,), jnp.int32),
        ],
    )
    return f(starts, ends, y_p).reshape(G, OUT)


def kernel(z, pos, batch, emb, Wm1, bm1, Wm2, bm2, Wc1, Wc2, bc2, Wl, bl,
           lin1_W, lin1_b, lin2_W, lin2_b):
    z = z.astype(jnp.int32)
    batch = batch.astype(jnp.int32)
    posf = pos.astype(jnp.float32)
    px = jnp.pad(posf[:, 0], (0, PPAD))
    py = jnp.pad(posf[:, 1], (0, PPAD))
    pz = jnp.pad(posf[:, 2], (0, PPAD))
    batch_t = batch.reshape(16, 512).T
    z2d = z.reshape(N, 1)
    embp = jnp.pad(emb, ((0, 28), (0, 0)))
    wm1p = jnp.pad(Wm1, ((0, 0), (0, NGP - NG), (0, 0)))
    bm1r = bm1.reshape(NI, 1, NF)
    bm2r = bm2.reshape(NI, 1, NF)

    tabs = _make_tables(wm1p, bm1r, Wm2, bm2r)
    starts2, ends2 = _make_bounds(batch_t)
    starts = starts2.reshape(G)
    ends = ends2.reshape(G)

    h, xl = _make_h0_xl(z2d, embp, Wc1[0])
    for i in range(NI):
        xl_p = jnp.pad(xl, ((0, PPAD), (0, 0))).reshape(-1)
        agg = _make_msg(starts, ends, px, py, pz, xl_p, tabs[i].reshape(-1))
        if i < NI - 1:
            h, xl = _make_h_update_xl(agg, h, Wc2[i], bc2[i].reshape(1, H),
                                      Wl[i], bl[i].reshape(1, H), Wc1[i + 1])
        else:
            y = _make_h_update_read(agg, h, Wc2[i], bc2[i].reshape(1, H),
                                    Wl[i], bl[i].reshape(1, H),
                                    lin1_W, lin1_b.reshape(1, H // 2),
                                    lin2_W, lin2_b.reshape(1, OUT))
    y_p = jnp.pad(y, ((0, 64), (0, 0))).reshape(-1)
    return _make_readout(starts, ends, y_p)


# speedup vs baseline: 31.9514x; 1.0001x over previous
import functools

import jax
import jax.numpy as jnp
from jax import lax
from jax.experimental import pallas as pl
from jax.experimental.pallas import tpu as pltpu
from jax.experimental.pallas import tpu_sc as plsc

N = 8192
G = 1024
H = 128
NF = 128
NG = 50
NGP = 64
OUT = 128
NI = 6
CUTOFF = 10.0
K = 512

NC = 2
NS = 16
NW = NC * NS
GPT = G // NW
CV = 32
CU = 32
SU = 384
VOFF = SU + 48
RT = 256
PPAD = SU + 64


def _ssp(x):
    return jax.nn.softplus(x) - jnp.log(2.0)


def _tables_body(wm1_ref, bm1_ref, wm2_ref, bm2_ref, t_ref):
    ki = lax.broadcasted_iota(jnp.int32, (K + 1, 1), 0)
    d = ki.astype(jnp.float32) * (CUTOFF / (K - 1))
    off = lax.broadcasted_iota(jnp.int32, (1, NGP), 1).astype(jnp.float32) \
        * (CUTOFF / (NG - 1))
    step = CUTOFF / (NG - 1)
    coeff = -0.5 / (step * step)
    ea = jnp.exp(coeff * (d - off) ** 2)
    t = _ssp(jnp.dot(ea, wm1_ref[0], preferred_element_type=jnp.float32)
             + bm1_ref[0])
    t = jnp.dot(t, wm2_ref[0], preferred_element_type=jnp.float32) + bm2_ref[0]
    c = 0.5 * (jnp.cos(d * (jnp.pi / CUTOFF)) + 1.0)
    c = jnp.where(ki < K, c, 0.0)
    t_ref[0] = t * c


def _make_tables(wm1p, bm1r, wm2, bm2r):
    return pl.pallas_call(
        _tables_body,
        grid=(NI,),
        in_specs=[
            pl.BlockSpec((1, NGP, NF), lambda i: (i, 0, 0)),
            pl.BlockSpec((1, 1, NF), lambda i: (i, 0, 0)),
            pl.BlockSpec((1, NF, NF), lambda i: (i, 0, 0)),
            pl.BlockSpec((1, 1, NF), lambda i: (i, 0, 0)),
        ],
        out_specs=pl.BlockSpec((1, K + 1, NF), lambda i: (i, 0, 0)),
        out_shape=jax.ShapeDtypeStruct((NI, K + 1, NF), jnp.float32),
    )(wm1p, bm1r, wm2, bm2r)


def _bounds_body(batch_ref, starts_ref, ends_ref):
    gv = lax.broadcasted_iota(jnp.int32, (1, G), 1)
    acc_lt = jnp.zeros((1, G), jnp.int32)
    acc_le = jnp.zeros((1, G), jnp.int32)
    for c in range(16):
        bc = batch_ref[:, pl.ds(c, 1)]
        acc_lt = acc_lt + jnp.sum((bc < gv).astype(jnp.int32), axis=0,
                                  keepdims=True)
        acc_le = acc_le + jnp.sum((bc <= gv).astype(jnp.int32), axis=0,
                                  keepdims=True)
    starts_ref[...] = acc_lt
    ends_ref[...] = acc_le


def _make_bounds(batch_t):
    return pl.pallas_call(
        _bounds_body,
        out_shape=(jax.ShapeDtypeStruct((1, G), jnp.int32),
                   jax.ShapeDtypeStruct((1, G), jnp.int32)),
    )(batch_t)


def _h0x_body(z_ref, emb_ref, wc1_ref, h_ref, xl_ref):
    zt = z_ref[...]
    iot = lax.broadcasted_iota(jnp.int32, (1, 128), 1)
    oneh = (zt == iot).astype(jnp.float32)
    h = jnp.dot(oneh, emb_ref[...], preferred_element_type=jnp.float32)
    h_ref[...] = h
    xl_ref[...] = jnp.dot(h, wc1_ref[...], preferred_element_type=jnp.float32)


def _make_h0_xl(z2d, embp, wc1):
    return pl.pallas_call(
        _h0x_body,
        grid=(N // RT,),
        in_specs=[
            pl.BlockSpec((RT, 1), lambda i: (i, 0)),
            pl.BlockSpec((128, 128), lambda i: (0, 0)),
            pl.BlockSpec((H, NF), lambda i: (0, 0)),
        ],
        out_specs=[
            pl.BlockSpec((RT, 128), lambda i: (i, 0)),
            pl.BlockSpec((RT, NF), lambda i: (i, 0)),
        ],
        out_shape=[jax.ShapeDtypeStruct((N, 128), jnp.float32),
                   jax.ShapeDtypeStruct((N, NF), jnp.float32)],
    )(z2d, embp, wc1)


def _khx_body(agg_ref, h_ref, wc2_ref, bc2_ref, wl_ref, bl_ref, wc1_ref,
              hout_ref, xl_ref):
    x2 = jnp.dot(agg_ref[...], wc2_ref[...],
                 preferred_element_type=jnp.float32) + bc2_ref[...]
    x2 = _ssp(x2)
    x2 = jnp.dot(x2, wl_ref[...], preferred_element_type=jnp.float32) \
        + bl_ref[...]
    h = h_ref[...] + x2
    hout_ref[...] = h
    xl_ref[...] = jnp.dot(h, wc1_ref[...], preferred_element_type=jnp.float32)


def _make_h_update_xl(agg, h, wc2, bc2r, wl, blr, wc1n):
    return pl.pallas_call(
        _khx_body,
        grid=(N // RT,),
        in_specs=[
            pl.BlockSpec((RT, NF), lambda i: (i, 0)),
            pl.BlockSpec((RT, H), lambda i: (i, 0)),
            pl.BlockSpec((NF, H), lambda i: (0, 0)),
            pl.BlockSpec((1, H), lambda i: (0, 0)),
            pl.BlockSpec((H, H), lambda i: (0, 0)),
            pl.BlockSpec((1, H), lambda i: (0, 0)),
            pl.BlockSpec((H, NF), lambda i: (0, 0)),
        ],
        out_specs=[
            pl.BlockSpec((RT, H), lambda i: (i, 0)),
            pl.BlockSpec((RT, NF), lambda i: (i, 0)),
        ],
        out_shape=[jax.ShapeDtypeStruct((N, H), jnp.float32),
                   jax.ShapeDtypeStruct((N, NF), jnp.float32)],
    )(agg, h, wc2, bc2r, wl, blr, wc1n)


def _khread_body(agg_ref, h_ref, wc2_ref, bc2_ref, wl_ref, bl_ref,
                 w1_ref, b1_ref, w2_ref, b2_ref, y_ref):
    x2 = jnp.dot(agg_ref[...], wc2_ref[...],
                 preferred_element_type=jnp.float32) + bc2_ref[...]
    x2 = _ssp(x2)
    x2 = jnp.dot(x2, wl_ref[...], preferred_element_type=jnp.float32) \
        + bl_ref[...]
    h = h_ref[...] + x2
    t = _ssp(jnp.dot(h, w1_ref[...],
                     preferred_element_type=jnp.float32) + b1_ref[...])
    y_ref[...] = jnp.dot(t, w2_ref[...],
                         preferred_element_type=jnp.float32) + b2_ref[...]


def _make_h_update_read(agg, h, wc2, bc2r, wl, blr, w1, b1r, w2, b2r):
    return pl.pallas_call(
        _khread_body,
        grid=(N // RT,),
        in_specs=[
            pl.BlockSpec((RT, NF), lambda i: (i, 0)),
            pl.BlockSpec((RT, H), lambda i: (i, 0)),
            pl.BlockSpec((NF, H), lambda i: (0, 0)),
            pl.BlockSpec((1, H), lambda i: (0, 0)),
            pl.BlockSpec((H, H), lambda i: (0, 0)),
            pl.BlockSpec((1, H), lambda i: (0, 0)),
            pl.BlockSpec((H, H // 2), lambda i: (0, 0)),
            pl.BlockSpec((1, H // 2), lambda i: (0, 0)),
            pl.BlockSpec((H // 2, OUT), lambda i: (0, 0)),
            pl.BlockSpec((1, OUT), lambda i: (0, 0)),
        ],
        out_specs=pl.BlockSpec((RT, OUT), lambda i: (i, 0)),
        out_shape=jax.ShapeDtypeStruct((N, OUT), jnp.float32),
    )(agg, h, wc2, bc2r, wl, blr, w1, b1r, w2, b2r)


def _sext(buf, i):
    return buf[pl.ds(i, 16)][0]


def _msg_body(starts_hbm, ends_hbm, px_hbm, py_hbm, pz_hbm, xl_hbm, tab_hbm,
              agg_hbm, tab_v, xlw, agg_v, pwx, pwy, pwz,
              sv, ev, sem_s, sem_w):
    wid = lax.axis_index("s") * NC + lax.axis_index("c")
    base = wid * GPT
    pltpu.sync_copy(tab_hbm, tab_v)
    pltpu.sync_copy(starts_hbm.at[pl.ds(base, GPT)], sv.at[pl.ds(0, GPT)])
    pltpu.sync_copy(ends_hbm.at[pl.ds(base, GPT)], ev.at[pl.ds(0, GPT)])
    lanes = lax.iota(jnp.int32, 16)
    zero16 = jnp.zeros((16,), jnp.float32)

    w0 = _sext(sv, 0)
    w1 = _sext(ev, GPT - 1)
    wlen = w1 - w0
    whole = wlen <= SU
    aw0 = (w0 // 8) * 8

    @pl.when(whole)
    def _stage_window():
        a1 = pltpu.async_copy(xl_hbm.at[pl.ds(w0 * NF, (SU + 16) * NF)],
                              xlw, sem_s)
        a2 = pltpu.async_copy(px_hbm.at[pl.ds(aw0, SU + 16)],
                              pwx.at[pl.ds(0, SU + 16)], sem_s)
        a3 = pltpu.async_copy(py_hbm.at[pl.ds(aw0, SU + 16)],
                              pwy.at[pl.ds(0, SU + 16)], sem_s)
        a4 = pltpu.async_copy(pz_hbm.at[pl.ds(aw0, SU + 16)],
                              pwz.at[pl.ds(0, SU + 16)], sem_s)
        a1.wait()
        a2.wait()
        a3.wait()
        a4.wait()

    def graph_body(gl, _):
        s = _sext(sv, gl)
        e = _sext(ev, gl)
        n = e - s

        def vc_body(vc, _):
            v0 = s + vc * CV
            nv = jnp.minimum(CV, n - vc * CV)
            av0 = (v0 // 8) * 8
            voff = v0 - av0

            @pl.when(jnp.logical_not(whole))
            def _stage_vpos():
                d1 = pltpu.async_copy(px_hbm.at[pl.ds(av0, CV + 8)],
                                      pwx.at[pl.ds(VOFF, CV + 8)], sem_s)
                d2 = pltpu.async_copy(py_hbm.at[pl.ds(av0, CV + 8)],
                                      pwy.at[pl.ds(VOFF, CV + 8)], sem_s)
                d3 = pltpu.async_copy(pz_hbm.at[pl.ds(av0, CV + 8)],
                                      pwz.at[pl.ds(VOFF, CV + 8)], sem_s)
                d1.wait()
                d2.wait()
                d3.wait()

            def z_body(ivz, _):
                for c2 in range(8):
                    agg_v[pl.ds(ivz * NF + 16 * c2, 16)] = zero16
                return 0

            lax.fori_loop(0, nv, z_body, 0)

            nuseg = jnp.where(whole, 1, (n + SU - 1) // SU)

            def useg_body(useg, _):
                us0 = s + useg * SU
                nu = jnp.minimum(e, us0 + SU) - us0

                @pl.when(jnp.logical_not(whole))
                def _stage_useg():
                    asu_c = (us0 // 8) * 8
                    e1 = pltpu.async_copy(
                        xl_hbm.at[pl.ds(us0 * NF, (SU + 16) * NF)], xlw,
                        sem_s)
                    e2 = pltpu.async_copy(px_hbm.at[pl.ds(asu_c, SU + 16)],
                                          pwx.at[pl.ds(0, SU + 16)], sem_s)
                    e3 = pltpu.async_copy(py_hbm.at[pl.ds(asu_c, SU + 16)],
                                          pwy.at[pl.ds(0, SU + 16)], sem_s)
                    e4 = pltpu.async_copy(pz_hbm.at[pl.ds(asu_c, SU + 16)],
                                          pwz.at[pl.ds(0, SU + 16)], sem_s)
                    e1.wait()
                    e2.wait()
                    e3.wait()
                    e4.wait()

                asu = jnp.where(whole, aw0, (us0 // 8) * 8)
                pbase = us0 - asu
                xbase0 = jnp.where(whole, us0 - w0, 0)
                nut = (nu + 15) // 16
                vwi = jnp.where(whole, jnp.clip(v0 - aw0, 0, SU + 6),
                                VOFF + voff)

                def vq_body(ivq):
                    iv0 = 2 * ivq
                    vs = [v0 + iv0 + q for q in range(2)]
                    vxs = [jnp.full((16,), _sext(pwx, vwi + iv0 + q))
                           for q in range(2)]
                    vys = [jnp.full((16,), _sext(pwy, vwi + iv0 + q))
                           for q in range(2)]
                    vzs = [jnp.full((16,), _sext(pwz, vwi + iv0 + q))
                           for q in range(2)]
                    accs = tuple(
                        agg_v[pl.ds((iv0 + q) * NF + 16 * c2, 16)]
                        for q in range(2) for c2 in range(8))

                    def ut_body(ut, accs):
                        lane0 = 16 * ut
                        px = pwx[pl.ds(pbase + lane0, 16)]
                        py = pwy[pl.ds(pbase + lane0, 16)]
                        pz = pwz[pl.ds(pbase + lane0, 16)]
                        ul = lane0 + lanes
                        lane_ok = ul < nu
                        uglob = us0 + ul
                        kis = []
                        for q in range(2):
                            dx = px - vxs[q]
                            dy = py - vys[q]
                            dz = pz - vzs[q]
                            d2 = dx * dx + dy * dy + dz * dz
                            d2 = jnp.where(lane_ok, d2, zero16)
                            sel = lane_ok & (d2 <= CUTOFF * CUTOFF) \
                                & (uglob != vs[q])
                            x = jnp.maximum(d2, 1e-24)
                            bits = lax.bitcast_convert_type(x, jnp.int32)
                            bits = 0x1FBD1DF5 + (bits >> 1)
                            y = lax.bitcast_convert_type(bits, jnp.float32)
                            y = 0.5 * (y + x / y)
                            y = 0.5 * (y + x / y)
                            kf = jnp.clip(y * ((K - 1) / CUTOFF) + 0.5,
                                          0.0, float(K - 1))
                            ki = kf.astype(jnp.int32) * NF
                            kis.append(jnp.where(sel, ki, K * NF))
                        acc_l = list(accs)
                        for j in range(16):
                            kjs = [kis[q][j] for q in range(2)]
                            xbase = (xbase0 + lane0 + j) * NF
                            for c2 in range(8):
                                xvec = xlw[pl.ds(xbase + 16 * c2, 16)]
                                for q in range(2):
                                    tvec = tab_v[pl.ds(kjs[q] + 16 * c2, 16)]
                                    acc_l[8 * q + c2] = \
                                        acc_l[8 * q + c2] + tvec * xvec
                        return tuple(acc_l)

                    accs = plsc.parallel_loop(0, nut, carry=accs,
                                              unroll=2)(ut_body)
                    for q in range(2):
                        for c2 in range(8):
                            agg_v[pl.ds((iv0 + q) * NF + 16 * c2, 16)] = \
                                accs[8 * q + c2]

                plsc.parallel_loop(0, (nv + 1) // 2, unroll=1)(vq_body)
                return 0

            lax.fori_loop(0, nuseg, useg_body, 0)

            def w_issue(iv, _):
                pltpu.async_copy(agg_v.at[pl.ds(iv * NF, NF)],
                                 agg_hbm.at[pl.ds((v0 + iv) * NF, NF)],
                                 sem_w)
                return 0

            lax.fori_loop(0, nv, w_issue, 0)

            def w_drain(iv, _):
                pltpu.make_async_copy(
                    agg_hbm.at[pl.ds(0, NF)], agg_v.at[pl.ds(0, NF)],
                    sem_w).wait()
                return 0

            lax.fori_loop(0, nv, w_drain, 0)
            return 0

        nvc = (n + CV - 1) // CV
        lax.fori_loop(0, nvc, vc_body, 0)
        return 0

    lax.fori_loop(0, GPT, graph_body, 0)


def _make_msg(starts, ends, px, py, pz, xl_p, tab):
    mesh = plsc.VectorSubcoreMesh(core_axis_name="c", subcore_axis_name="s",
                                  num_cores=NC, num_subcores=NS)
    f = pl.kernel(
        _msg_body,
        out_type=jax.ShapeDtypeStruct((N * NF,), jnp.float32),
        mesh=mesh,
        scratch_types=[
            pltpu.VMEM(((K + 1) * NF,), jnp.float32),
            pltpu.VMEM(((SU + 16) * NF,), jnp.float32),
            pltpu.VMEM((CV * NF,), jnp.float32),
            pltpu.VMEM((SU + 112,), jnp.float32),
            pltpu.VMEM((SU + 112,), jnp.float32),
            pltpu.VMEM((SU + 112,), jnp.float32),
            pltpu.VMEM((GPT + 16,), jnp.int32),
            pltpu.VMEM((GPT + 16,), jnp.int32),
            pltpu.SemaphoreType.DMA,
            pltpu.SemaphoreType.DMA,
        ],
    )
    return f(starts, ends, px, py, pz, xl_p, tab).reshape(N, NF)


def _readout_body(starts_hbm, ends_hbm, y_hbm, out_hbm, y_v, row_v, sv, ev):
    wid = lax.axis_index("s") * NC + lax.axis_index("c")
    base = wid * GPT
    pltpu.sync_copy(starts_hbm.at[pl.ds(base, GPT)], sv.at[pl.ds(0, GPT)])
    pltpu.sync_copy(ends_hbm.at[pl.ds(base, GPT)], ev.at[pl.ds(0, GPT)])
    zero16 = jnp.zeros((16,), jnp.float32)

    def graph_body(gl, _):
        s = _sext(sv, gl)
        e = _sext(ev, gl)
        n = e - s
        nuc = (n + CU - 1) // CU

        def uc_body(uc, accs):
            u0 = s + uc * CU
            nu = jnp.minimum(CU, n - uc * CU)
            pltpu.sync_copy(y_hbm.at[pl.ds(u0 * OUT, CU * OUT)], y_v)

            def r_body(r, accs):
                return tuple(accs[c2] + y_v[pl.ds(r * OUT + 16 * c2, 16)]
                             for c2 in range(8))

            return lax.fori_loop(0, nu, r_body, accs)

        accs = lax.fori_loop(0, nuc, uc_body, tuple(zero16 for _ in range(8)))
        for c2 in range(8):
            row_v[pl.ds(16 * c2, 16)] = accs[c2]
        pltpu.sync_copy(row_v, out_hbm.at[pl.ds((base + gl) * OUT, OUT)])
        return 0

    lax.fori_loop(0, GPT, graph_body, 0)


def _make_readout(starts, ends, y_p):
    mesh = plsc.VectorSubcoreMesh(core_axis_name="c", subcore_axis_name="s",
                                  num_cores=NC, num_subcores=NS)
    f = pl.kernel(
        _readout_body,
        out_type=jax.ShapeDtypeStruct((G * OUT,), jnp.float32),
        mesh=mesh,
        scratch_types=[
            pltpu.VMEM((CU * OUT,), jnp.float32),
            pltpu.VMEM((OUT,), jnp.float32),
            pltpu.VMEM((GPT + 16,), jnp.int32),
            pltpu.VMEM((GPT + 16,), jnp.int32),
        ],
    )
    return f(starts, ends, y_p).reshape(G, OUT)


def kernel(z, pos, batch, emb, Wm1, bm1, Wm2, bm2, Wc1, Wc2, bc2, Wl, bl,
           lin1_W, lin1_b, lin2_W, lin2_b):
    z = z.astype(jnp.int32)
    batch = batch.astype(jnp.int32)
    posf = pos.astype(jnp.float32)
    px = jnp.pad(posf[:, 0], (0, PPAD))
    py = jnp.pad(posf[:, 1], (0, PPAD))
    pz = jnp.pad(posf[:, 2], (0, PPAD))
    batch_t = batch.reshape(16, 512).T
    z2d = z.reshape(N, 1)
    embp = jnp.pad(emb, ((0, 28), (0, 0)))
    wm1p = jnp.pad(Wm1, ((0, 0), (0, NGP - NG), (0, 0)))
    bm1r = bm1.reshape(NI, 1, NF)
    bm2r = bm2.reshape(NI, 1, NF)

    tabs = _make_tables(wm1p, bm1r, Wm2, bm2r)
    starts2, ends2 = _make_bounds(batch_t)
    starts = starts2.reshape(G)
    ends = ends2.reshape(G)

    h, xl = _make_h0_xl(z2d, embp, Wc1[0])
    for i in range(NI):
        xl_p = jnp.pad(xl, ((0, PPAD), (0, 0))).reshape(-1)
        agg = _make_msg(starts, ends, px, py, pz, xl_p, tabs[i].reshape(-1))
        if i < NI - 1:
            h, xl = _make_h_update_xl(agg, h, Wc2[i], bc2[i].reshape(1, H),
                                      Wl[i], bl[i].reshape(1, H), Wc1[i + 1])
        else:
            y = _make_h_update_read(agg, h, Wc2[i], bc2[i].reshape(1, H),
                                    Wl[i], bl[i].reshape(1, H),
                                    lin1_W, lin1_b.reshape(1, H // 2),
                                    lin2_W, lin2_b.reshape(1, OUT))
    y_p = jnp.pad(y, ((0, 64), (0, 0))).reshape(-1)
    return _make_readout(starts, ends, y_p)


# speedup vs baseline: 31.9576x; 1.0002x over previous
import jax
import jax.numpy as jnp
from jax import lax
from jax.experimental import pallas as pl
from jax.experimental.pallas import tpu as pltpu
from jax.experimental.pallas import tpu_sc as plsc

N = 8192
G = 1024
H = 128
NF = 128
NG = 50
NGP = 64
OUT = 128
NI = 6
CUTOFF = 10.0
K = 512

NC = 2
NS = 16
NW = NC * NS
GPT = G // NW
CV = 32
CU = 32
SU = 384
VOFF = SU + 48
RT = 256
PPAD = SU + 64


def _ssp(x):
    return jax.nn.softplus(x) - jnp.log(2.0)


def _tables_body(wm1_ref, bm1_ref, wm2_ref, bm2_ref, t_ref):
    ki = lax.broadcasted_iota(jnp.int32, (K + 1, 1), 0)
    d = ki.astype(jnp.float32) * (CUTOFF / (K - 1))
    off = lax.broadcasted_iota(jnp.int32, (1, NGP), 1).astype(jnp.float32) \
        * (CUTOFF / (NG - 1))
    step = CUTOFF / (NG - 1)
    coeff = -0.5 / (step * step)
    ea = jnp.exp(coeff * (d - off) ** 2)
    t = _ssp(jnp.dot(ea, wm1_ref[0], preferred_element_type=jnp.float32)
             + bm1_ref[0])
    t = jnp.dot(t, wm2_ref[0], preferred_element_type=jnp.float32) + bm2_ref[0]
    c = 0.5 * (jnp.cos(d * (jnp.pi / CUTOFF)) + 1.0)
    c = jnp.where(ki < K, c, 0.0)
    t_ref[0] = t * c


def _make_tables(wm1p, bm1r, wm2, bm2r):
    return pl.pallas_call(
        _tables_body,
        grid=(NI,),
        in_specs=[
            pl.BlockSpec((1, NGP, NF), lambda i: (i, 0, 0)),
            pl.BlockSpec((1, 1, NF), lambda i: (i, 0, 0)),
            pl.BlockSpec((1, NF, NF), lambda i: (i, 0, 0)),
            pl.BlockSpec((1, 1, NF), lambda i: (i, 0, 0)),
        ],
        out_specs=pl.BlockSpec((1, K + 1, NF), lambda i: (i, 0, 0)),
        out_shape=jax.ShapeDtypeStruct((NI, K + 1, NF), jnp.float32),
    )(wm1p, bm1r, wm2, bm2r)


def _bounds_body(batch_ref, starts_ref, ends_ref):
    gv = lax.broadcasted_iota(jnp.int32, (1, G), 1)
    acc_lt = jnp.zeros((1, G), jnp.int32)
    acc_le = jnp.zeros((1, G), jnp.int32)
    for c in range(16):
        bc = batch_ref[:, pl.ds(c, 1)]
        acc_lt = acc_lt + jnp.sum((bc < gv).astype(jnp.int32), axis=0,
                                  keepdims=True)
        acc_le = acc_le + jnp.sum((bc <= gv).astype(jnp.int32), axis=0,
                                  keepdims=True)
    starts_ref[...] = acc_lt
    ends_ref[...] = acc_le


def _make_bounds(batch_t):
    return pl.pallas_call(
        _bounds_body,
        out_shape=(jax.ShapeDtypeStruct((1, G), jnp.int32),
                   jax.ShapeDtypeStruct((1, G), jnp.int32)),
    )(batch_t)


def _h0x_body(z_ref, emb_ref, wc1_ref, h_ref, xl_ref):
    zt = z_ref[...]
    iot = lax.broadcasted_iota(jnp.int32, (1, 128), 1)
    oneh = (zt == iot).astype(jnp.float32)
    h = jnp.dot(oneh, emb_ref[...], preferred_element_type=jnp.float32)
    h_ref[...] = h
    xl_ref[...] = jnp.dot(h, wc1_ref[...], preferred_element_type=jnp.float32)


def _make_h0_xl(z2d, embp, wc1):
    return pl.pallas_call(
        _h0x_body,
        grid=(N // RT,),
        in_specs=[
            pl.BlockSpec((RT, 1), lambda i: (i, 0)),
            pl.BlockSpec((128, 128), lambda i: (0, 0)),
            pl.BlockSpec((H, NF), lambda i: (0, 0)),
        ],
        out_specs=[
            pl.BlockSpec((RT, 128), lambda i: (i, 0)),
            pl.BlockSpec((RT, NF), lambda i: (i, 0)),
        ],
        out_shape=[jax.ShapeDtypeStruct((N, 128), jnp.float32),
                   jax.ShapeDtypeStruct((N, NF), jnp.float32)],
    )(z2d, embp, wc1)


def _khx_body(agg_ref, h_ref, wc2_ref, bc2_ref, wl_ref, bl_ref, wc1_ref,
              hout_ref, xl_ref):
    x2 = jnp.dot(agg_ref[...], wc2_ref[...],
                 preferred_element_type=jnp.float32) + bc2_ref[...]
    x2 = _ssp(x2)
    x2 = jnp.dot(x2, wl_ref[...], preferred_element_type=jnp.float32) \
        + bl_ref[...]
    h = h_ref[...] + x2
    hout_ref[...] = h
    xl_ref[...] = jnp.dot(h, wc1_ref[...], preferred_element_type=jnp.float32)


def _make_h_update_xl(agg, h, wc2, bc2r, wl, blr, wc1n):
    return pl.pallas_call(
        _khx_body,
        grid=(N // RT,),
        in_specs=[
            pl.BlockSpec((RT, NF), lambda i: (i, 0)),
            pl.BlockSpec((RT, H), lambda i: (i, 0)),
            pl.BlockSpec((NF, H), lambda i: (0, 0)),
            pl.BlockSpec((1, H), lambda i: (0, 0)),
            pl.BlockSpec((H, H), lambda i: (0, 0)),
            pl.BlockSpec((1, H), lambda i: (0, 0)),
            pl.BlockSpec((H, NF), lambda i: (0, 0)),
        ],
        out_specs=[
            pl.BlockSpec((RT, H), lambda i: (i, 0)),
            pl.BlockSpec((RT, NF), lambda i: (i, 0)),
        ],
        out_shape=[jax.ShapeDtypeStruct((N, H), jnp.float32),
                   jax.ShapeDtypeStruct((N, NF), jnp.float32)],
    )(agg, h, wc2, bc2r, wl, blr, wc1n)


def _khread_body(agg_ref, h_ref, wc2_ref, bc2_ref, wl_ref, bl_ref,
                 w1_ref, b1_ref, w2_ref, b2_ref, y_ref):
    x2 = jnp.dot(agg_ref[...], wc2_ref[...],
                 preferred_element_type=jnp.float32) + bc2_ref[...]
    x2 = _ssp(x2)
    x2 = jnp.dot(x2, wl_ref[...], preferred_element_type=jnp.float32) \
        + bl_ref[...]
    h = h_ref[...] + x2
    t = _ssp(jnp.dot(h, w1_ref[...],
                     preferred_element_type=jnp.float32) + b1_ref[...])
    y_ref[...] = jnp.dot(t, w2_ref[...],
                         preferred_element_type=jnp.float32) + b2_ref[...]


def _make_h_update_read(agg, h, wc2, bc2r, wl, blr, w1, b1r, w2, b2r):
    return pl.pallas_call(
        _khread_body,
        grid=(N // RT,),
        in_specs=[
            pl.BlockSpec((RT, NF), lambda i: (i, 0)),
            pl.BlockSpec((RT, H), lambda i: (i, 0)),
            pl.BlockSpec((NF, H), lambda i: (0, 0)),
            pl.BlockSpec((1, H), lambda i: (0, 0)),
            pl.BlockSpec((H, H), lambda i: (0, 0)),
            pl.BlockSpec((1, H), lambda i: (0, 0)),
            pl.BlockSpec((H, H // 2), lambda i: (0, 0)),
            pl.BlockSpec((1, H // 2), lambda i: (0, 0)),
            pl.BlockSpec((H // 2, OUT), lambda i: (0, 0)),
            pl.BlockSpec((1, OUT), lambda i: (0, 0)),
        ],
        out_specs=pl.BlockSpec((RT, OUT), lambda i: (i, 0)),
        out_shape=jax.ShapeDtypeStruct((N, OUT), jnp.float32),
    )(agg, h, wc2, bc2r, wl, blr, w1, b1r, w2, b2r)


def _sext(buf, i):
    return buf[pl.ds(i, 16)][0]


def _msg_body(starts_hbm, ends_hbm, px_hbm, py_hbm, pz_hbm, xl_hbm, tab_hbm,
              agg_hbm, tab_v, xlw, agg_v, pwx, pwy, pwz,
              sv, ev, sem_s, sem_w):
    wid = lax.axis_index("s") * NC + lax.axis_index("c")
    base = wid * GPT
    pltpu.sync_copy(tab_hbm, tab_v)
    pltpu.sync_copy(starts_hbm.at[pl.ds(base, GPT)], sv.at[pl.ds(0, GPT)])
    pltpu.sync_copy(ends_hbm.at[pl.ds(base, GPT)], ev.at[pl.ds(0, GPT)])
    lanes = lax.iota(jnp.int32, 16)
    zero16 = jnp.zeros((16,), jnp.float32)

    w0 = _sext(sv, 0)
    w1 = _sext(ev, GPT - 1)
    wlen = w1 - w0
    whole = wlen <= SU
    aw0 = (w0 // 8) * 8

    @pl.when(whole)
    def _stage_window():
        a1 = pltpu.async_copy(xl_hbm.at[pl.ds(w0 * NF, (SU + 16) * NF)],
                              xlw, sem_s)
        a2 = pltpu.async_copy(px_hbm.at[pl.ds(aw0, SU + 16)],
                              pwx.at[pl.ds(0, SU + 16)], sem_s)
        a3 = pltpu.async_copy(py_hbm.at[pl.ds(aw0, SU + 16)],
                              pwy.at[pl.ds(0, SU + 16)], sem_s)
        a4 = pltpu.async_copy(pz_hbm.at[pl.ds(aw0, SU + 16)],
                              pwz.at[pl.ds(0, SU + 16)], sem_s)
        a1.wait()
        a2.wait()
        a3.wait()
        a4.wait()

    def graph_body(gl, _):
        s = _sext(sv, gl)
        e = _sext(ev, gl)
        n = e - s

        def vc_body(vc, _):
            v0 = s + vc * CV
            nv = jnp.minimum(CV, n - vc * CV)
            av0 = (v0 // 8) * 8
            voff = v0 - av0

            @pl.when(jnp.logical_not(whole))
            def _stage_vpos():
                d1 = pltpu.async_copy(px_hbm.at[pl.ds(av0, CV + 8)],
                                      pwx.at[pl.ds(VOFF, CV + 8)], sem_s)
                d2 = pltpu.async_copy(py_hbm.at[pl.ds(av0, CV + 8)],
                                      pwy.at[pl.ds(VOFF, CV + 8)], sem_s)
                d3 = pltpu.async_copy(pz_hbm.at[pl.ds(av0, CV + 8)],
                                      pwz.at[pl.ds(VOFF, CV + 8)], sem_s)
                d1.wait()
                d2.wait()
                d3.wait()

            def z_body(ivz, _):
                for c2 in range(8):
                    agg_v[pl.ds(ivz * NF + 16 * c2, 16)] = zero16
                return 0

            lax.fori_loop(0, nv, z_body, 0)

            nuseg = jnp.where(whole, 1, (n + SU - 1) // SU)

            def useg_body(useg, _):
                us0 = s + useg * SU
                nu = jnp.minimum(e, us0 + SU) - us0

                @pl.when(jnp.logical_not(whole))
                def _stage_useg():
                    asu_c = (us0 // 8) * 8
                    e1 = pltpu.async_copy(
                        xl_hbm.at[pl.ds(us0 * NF, (SU + 16) * NF)], xlw,
                        sem_s)
                    e2 = pltpu.async_copy(px_hbm.at[pl.ds(asu_c, SU + 16)],
                                          pwx.at[pl.ds(0, SU + 16)], sem_s)
                    e3 = pltpu.async_copy(py_hbm.at[pl.ds(asu_c, SU + 16)],
                                          pwy.at[pl.ds(0, SU + 16)], sem_s)
                    e4 = pltpu.async_copy(pz_hbm.at[pl.ds(asu_c, SU + 16)],
                                          pwz.at[pl.ds(0, SU + 16)], sem_s)
                    e1.wait()
                    e2.wait()
                    e3.wait()
                    e4.wait()

                asu = jnp.where(whole, aw0, (us0 // 8) * 8)
                pbase = us0 - asu
                xbase0 = jnp.where(whole, us0 - w0, 0)
                nut = (nu + 15) // 16
                vwi = jnp.where(whole, jnp.clip(v0 - aw0, 0, SU + 6),
                                VOFF + voff)

                def vq_body(ivq):
                    iv0 = 2 * ivq
                    vs = [v0 + iv0 + q for q in range(2)]
                    vxs = [jnp.full((16,), _sext(pwx, vwi + iv0 + q))
                           for q in range(2)]
                    vys = [jnp.full((16,), _sext(pwy, vwi + iv0 + q))
                           for q in range(2)]
                    vzs = [jnp.full((16,), _sext(pwz, vwi + iv0 + q))
                           for q in range(2)]
                    accs = tuple(
                        agg_v[pl.ds((iv0 + q) * NF + 16 * c2, 16)]
                        for q in range(2) for c2 in range(8))

                    def ut_body(ut, accs):
                        lane0 = 16 * ut
                        px = pwx[pl.ds(pbase + lane0, 16)]
                        py = pwy[pl.ds(pbase + lane0, 16)]
                        pz = pwz[pl.ds(pbase + lane0, 16)]
                        ul = lane0 + lanes
                        lane_ok = ul < nu
                        uglob = us0 + ul
                        kis = []
                        for q in range(2):
                            dx = px - vxs[q]
                            dy = py - vys[q]
                            dz = pz - vzs[q]
                            d2 = dx * dx + dy * dy + dz * dz
                            d2 = jnp.where(lane_ok, d2, zero16)
                            sel = lane_ok & (d2 <= CUTOFF * CUTOFF) \
                                & (uglob != vs[q])
                            x = jnp.maximum(d2, 1e-24)
                            bits = lax.bitcast_convert_type(x, jnp.int32)
                            bits = 0x1FBD1DF5 + (bits >> 1)
                            y = lax.bitcast_convert_type(bits, jnp.float32)
                            y = 0.5 * (y + x / y)
                            y = 0.5 * (y + x / y)
                            kf = jnp.clip(y * ((K - 1) / CUTOFF) + 0.5,
                                          0.0, float(K - 1))
                            ki = kf.astype(jnp.int32) * NF
                            kis.append(jnp.where(sel, ki, K * NF))
                        acc_l = list(accs)
                        for j in range(16):
                            kjs = [kis[q][j] for q in range(2)]
                            xbase = (xbase0 + lane0 + j) * NF
                            for c2 in range(8):
                                xvec = xlw[pl.ds(xbase + 16 * c2, 16)]
                                for q in range(2):
                                    tvec = tab_v[pl.ds(kjs[q] + 16 * c2, 16)]
                                    acc_l[8 * q + c2] = \
                                        acc_l[8 * q + c2] + tvec * xvec
                        return tuple(acc_l)

                    accs = plsc.parallel_loop(0, nut, carry=accs,
                                              unroll=2)(ut_body)
                    for q in range(2):
                        for c2 in range(8):
                            agg_v[pl.ds((iv0 + q) * NF + 16 * c2, 16)] = \
                                accs[8 * q + c2]

                plsc.parallel_loop(0, (nv + 1) // 2, unroll=1)(vq_body)
                return 0

            lax.fori_loop(0, nuseg, useg_body, 0)

            def w_issue(iv, _):
                pltpu.async_copy(agg_v.at[pl.ds(iv * NF, NF)],
                                 agg_hbm.at[pl.ds((v0 + iv) * NF, NF)],
                                 sem_w)
                return 0

            lax.fori_loop(0, nv, w_issue, 0)

            def w_drain(iv, _):
                pltpu.make_async_copy(
                    agg_hbm.at[pl.ds(0, NF)], agg_v.at[pl.ds(0, NF)],
                    sem_w).wait()
                return 0

            lax.fori_loop(0, nv, w_drain, 0)
            return 0

        nvc = (n + CV - 1) // CV
        lax.fori_loop(0, nvc, vc_body, 0)
        return 0

    lax.fori_loop(0, GPT, graph_body, 0)


def _make_msg(starts, ends, px, py, pz, xl_p, tab):
    mesh = plsc.VectorSubcoreMesh(core_axis_name="c", subcore_axis_name="s",
                                  num_cores=NC, num_subcores=NS)
    f = pl.kernel(
        _msg_body,
        out_type=jax.ShapeDtypeStruct((N * NF,), jnp.float32),
        mesh=mesh,
        scratch_types=[
            pltpu.VMEM(((K + 1) * NF,), jnp.float32),
            pltpu.VMEM(((SU + 16) * NF,), jnp.float32),
            pltpu.VMEM((CV * NF,), jnp.float32),
            pltpu.VMEM((SU + 112,), jnp.float32),
            pltpu.VMEM((SU + 112,), jnp.float32),
            pltpu.VMEM((SU + 112,), jnp.float32),
            pltpu.VMEM((GPT + 16,), jnp.int32),
            pltpu.VMEM((GPT + 16,), jnp.int32),
            pltpu.SemaphoreType.DMA,
            pltpu.SemaphoreType.DMA,
        ],
    )
    return f(starts, ends, px, py, pz, xl_p, tab).reshape(N, NF)


def _readout_body(starts_hbm, ends_hbm, y_hbm, out_hbm, y_v, row_v, sv, ev):
    wid = lax.axis_index("s") * NC + lax.axis_index("c")
    base = wid * GPT
    pltpu.sync_copy(starts_hbm.at[pl.ds(base, GPT)], sv.at[pl.ds(0, GPT)])
    pltpu.sync_copy(ends_hbm.at[pl.ds(base, GPT)], ev.at[pl.ds(0, GPT)])
    zero16 = jnp.zeros((16,), jnp.float32)

    def graph_body(gl, _):
        s = _sext(sv, gl)
        e = _sext(ev, gl)
        n = e - s
        nuc = (n + CU - 1) // CU

        def uc_body(uc, accs):
            u0 = s + uc * CU
            nu = jnp.minimum(CU, n - uc * CU)
            pltpu.sync_copy(y_hbm.at[pl.ds(u0 * OUT, CU * OUT)], y_v)

            def r_body(r, accs):
                return tuple(accs[c2] + y_v[pl.ds(r * OUT + 16 * c2, 16)]
                             for c2 in range(8))

            return lax.fori_loop(0, nu, r_body, accs)

        accs = lax.fori_loop(0, nuc, uc_body, tuple(zero16 for _ in range(8)))
        for c2 in range(8):
            row_v[pl.ds(16 * c2, 16)] = accs[c2]
        pltpu.sync_copy(row_v, out_hbm.at[pl.ds((base + gl) * OUT, OUT)])
        return 0

    lax.fori_loop(0, GPT, graph_body, 0)


def _make_readout(starts, ends, y_p):
    mesh = plsc.VectorSubcoreMesh(core_axis_name="c", subcore_axis_name="s",
                                  num_cores=NC, num_subcores=NS)
    f = pl.kernel(
        _readout_body,
        out_type=jax.ShapeDtypeStruct((G * OUT,), jnp.float32),
        mesh=mesh,
        scratch_types=[
            pltpu.VMEM((CU * OUT,), jnp.float32),
            pltpu.VMEM((OUT,), jnp.float32),
            pltpu.VMEM((GPT + 16,), jnp.int32),
            pltpu.VMEM((GPT + 16,), jnp.int32),
        ],
    )
    return f(starts, ends, y_p).reshape(G, OUT)


def kernel(z, pos, batch, emb, Wm1, bm1, Wm2, bm2, Wc1, Wc2, bc2, Wl, bl,
           lin1_W, lin1_b, lin2_W, lin2_b):
    z = z.astype(jnp.int32)
    batch = batch.astype(jnp.int32)
    posf = pos.astype(jnp.float32)
    px = jnp.pad(posf[:, 0], (0, PPAD))
    py = jnp.pad(posf[:, 1], (0, PPAD))
    pz = jnp.pad(posf[:, 2], (0, PPAD))
    batch_t = batch.reshape(16, 512).T
    z2d = z.reshape(N, 1)
    embp = jnp.pad(emb, ((0, 28), (0, 0)))
    wm1p = jnp.pad(Wm1, ((0, 0), (0, NGP - NG), (0, 0)))
    bm1r = bm1.reshape(NI, 1, NF)
    bm2r = bm2.reshape(NI, 1, NF)

    tabs = _make_tables(wm1p, bm1r, Wm2, bm2r)
    starts2, ends2 = _make_bounds(batch_t)
    starts = starts2.reshape(G)
    ends = ends2.reshape(G)

    h, xl = _make_h0_xl(z2d, embp, Wc1[0])
    for i in range(NI):
        xl_p = jnp.pad(xl, ((0, PPAD), (0, 0))).reshape(-1)
        agg = _make_msg(starts, ends, px, py, pz, xl_p, tabs[i].reshape(-1))
        if i < NI - 1:
            h, xl = _make_h_update_xl(agg, h, Wc2[i], bc2[i].reshape(1, H),
                                      Wl[i], bl[i].reshape(1, H), Wc1[i + 1])
        else:
            y = _make_h_update_read(agg, h, Wc2[i], bc2[i].reshape(1, H),
                                    Wl[i], bl[i].reshape(1, H),
                                    lin1_W, lin1_b.reshape(1, H // 2),
                                    lin2_W, lin2_b.reshape(1, OUT))
    y_p = jnp.pad(y, ((0, 64), (0, 0))).reshape(-1)
    return _make_readout(starts, ends, y_p)
